# Initial kernel scaffold; baseline (speedup 1.0000x reference)
import jax
import jax.numpy as jnp
from jax.experimental import pallas as pl


def kernel(node_features, edge_features, edge_index, edge_W1, edge_b1, edge_W2, edge_b2, edge_W3, edge_b3, node_W1, node_b1, node_W2, node_b2, node_W3, node_b3):
    raise NotImplementedError("write your pallas kernel here")


# speedup vs baseline: 2.7208x; 2.7208x over previous
import functools

import jax
import jax.numpy as jnp
from jax import lax
from jax.experimental import pallas as pl
from jax.experimental.pallas import tpu as pltpu
from jax.experimental.pallas import tpu_sc as plsc

N = 10000
E = 160000
D = 128
NC = 2
NS = 16
NW = NC * NS
CH = 128
ROWS = E // CH
RPW = -(-ROWS // NW)
NA = 10240
NPS = NA // NS


def _worker_nrows(wid):
    return jnp.where(wid < ROWS - (RPW - 1) * NW, RPW, RPW - 1)


@functools.cache
def _sc_kernels():
    mesh = plsc.VectorSubcoreMesh(core_axis_name="c", subcore_axis_name="s",
                                  num_cores=NC, num_subcores=NS)

    @functools.partial(
        pl.kernel,
        out_type=(jax.ShapeDtypeStruct((E, D), jnp.float32),
                  jax.ShapeDtypeStruct((E, D), jnp.float32)),
        mesh=mesh,
        scratch_types=[
            pltpu.VMEM((RPW, CH), jnp.int32),
            pltpu.VMEM((RPW, CH), jnp.int32),
            pltpu.VMEM((CH, D), jnp.float32),
            pltpu.VMEM((CH, D), jnp.float32),
            pltpu.SemaphoreType.DMA,
            pltpu.SemaphoreType.DMA,
        ],
    )
    def _gather_sc(ps_hbm, pd_hbm, src_hbm, dst_hbm, g1_hbm, g2_hbm,
                   sidx, didx, bufa, bufb, sema, semb):
        wid = lax.axis_index("s") * NC + lax.axis_index("c")
        pltpu.sync_copy(src_hbm.at[wid], sidx)
        pltpu.sync_copy(dst_hbm.at[wid], didx)

        def body(r, carry):
            row = r * NW + wid
            ca = pltpu.async_copy(ps_hbm.at[sidx.at[r]], bufa, sema)
            cb = pltpu.async_copy(pd_hbm.at[didx.at[r]], bufb, semb)
            ca.wait()
            cb.wait()
            pltpu.sync_copy(bufa, g1_hbm.at[pl.ds(row * CH, CH)])
            pltpu.sync_copy(bufb, g2_hbm.at[pl.ds(row * CH, CH)])
            return carry

        lax.fori_loop(0, _worker_nrows(wid), body, 0)

    @functools.partial(
        pl.kernel,
        out_type=jax.ShapeDtypeStruct((NC, NA, D), jnp.float32),
        mesh=mesh,
        scratch_types=[
            pltpu.VMEM((RPW, CH), jnp.int32),
            pltpu.VMEM((CH, D), jnp.float32),
            pltpu.VMEM_SHARED((NA, D), jnp.float32),
        ],
    )
    def _scatter_sc(ef_hbm, dst_hbm, zeros_hbm, out_hbm, didx, buf, acc):
        cid = lax.axis_index("c")
        sid = lax.axis_index("s")
        wid = sid * NC + cid

        pltpu.sync_copy(zeros_hbm, buf)
        def zinit(k, carry):
            pltpu.sync_copy(buf, acc.at[pl.ds(sid * NPS + k * CH, CH)])
            return carry
        lax.fori_loop(0, NPS // CH, zinit, 0)
        pltpu.sync_copy(dst_hbm.at[wid], didx)
        plsc.subcore_barrier()

        def body(r, carry):
            row = r * NW + wid
            pltpu.sync_copy(ef_hbm.at[pl.ds(row * CH, CH)], buf)
            pltpu.sync_copy(buf, acc.at[didx.at[r]], add=True)
            return carry

        lax.fori_loop(0, _worker_nrows(wid), body, 0)
        plsc.subcore_barrier()

        def wout(k, carry):
            pltpu.sync_copy(acc.at[pl.ds(sid * NPS + k * CH, CH)], buf)
            pltpu.sync_copy(buf, out_hbm.at[cid, pl.ds(sid * NPS + k * CH, CH)])
            return carry
        lax.fori_loop(0, NPS // CH, wout, 0)

    return _gather_sc, _scatter_sc


_BN = 1000
_BE = 640


def _project_body(nf, w1s, w1d, b1, ps, pd):
    x = nf[...]
    ps[...] = jnp.dot(x, w1s[...], preferred_element_type=jnp.float32) + b1[...]
    pd[...] = jnp.dot(x, w1d[...], preferred_element_type=jnp.float32)


def _edge_body(g1, g2, ef, w1e, w2, b2, w3, b3, out):
    x = ef[...]
    h = g1[...] + g2[...] + jnp.dot(x, w1e[...], preferred_element_type=jnp.float32)
    h = jnp.maximum(h, 0.0)
    h = jnp.dot(h, w2[...], preferred_element_type=jnp.float32) + b2[...]
    h = jnp.maximum(h, 0.0)
    out[...] = jnp.dot(h, w3[...], preferred_element_type=jnp.float32) + b3[...] + x


def _node_body(nf, a0, a1, w1a, w1b, b1, w2, b2, w3, b3, out):
    x = nf[...]
    agg = a0[...] + a1[...]
    h = (jnp.dot(x, w1a[...], preferred_element_type=jnp.float32)
         + jnp.dot(agg, w1b[...], preferred_element_type=jnp.float32) + b1[...])
    h = jnp.maximum(h, 0.0)
    h = jnp.dot(h, w2[...], preferred_element_type=jnp.float32) + b2[...]
    h = jnp.maximum(h, 0.0)
    out[...] = jnp.dot(h, w3[...], preferred_element_type=jnp.float32) + b3[...] + x


def _row_spec(block):
    return pl.BlockSpec((block, D), lambda b: (b, 0))


def _w_spec():
    return pl.BlockSpec((D, D), lambda b: (0, 0))


def _b_spec():
    return pl.BlockSpec((1, D), lambda b: (0, 0))


_project_tc = pl.pallas_call(
    _project_body,
    grid=(N // _BN,),
    in_specs=[_row_spec(_BN), _w_spec(), _w_spec(), _b_spec()],
    out_specs=[_row_spec(_BN), _row_spec(_BN)],
    out_shape=[jax.ShapeDtypeStruct((N, D), jnp.float32),
               jax.ShapeDtypeStruct((N, D), jnp.float32)],
)

_edge_tc = pl.pallas_call(
    _edge_body,
    grid=(E // _BE,),
    in_specs=[_row_spec(_BE), _row_spec(_BE), _row_spec(_BE),
              _w_spec(), _w_spec(), _b_spec(), _w_spec(), _b_spec()],
    out_specs=_row_spec(_BE),
    out_shape=jax.ShapeDtypeStruct((E, D), jnp.float32),
)

_node_tc = pl.pallas_call(
    _node_body,
    grid=(N // _BN,),
    in_specs=[_row_spec(_BN), _row_spec(_BN), _row_spec(_BN),
              _w_spec(), _w_spec(), _b_spec(), _w_spec(), _b_spec(),
              _w_spec(), _b_spec()],
    out_specs=_row_spec(_BN),
    out_shape=jax.ShapeDtypeStruct((N, D), jnp.float32),
)


def _worker_major(idx):
    pad = NW * RPW * CH - E
    idx = jnp.concatenate([idx, jnp.zeros((pad,), idx.dtype)])
    return idx.reshape(RPW, NW, CH).transpose(1, 0, 2)


@jax.jit
def kernel(node_features, edge_features, edge_index,
           edge_W1, edge_b1, edge_W2, edge_b2, edge_W3, edge_b3,
           node_W1, node_b1, node_W2, node_b2, node_W3, node_b3):
    gather_sc, scatter_sc = _sc_kernels()
    src3 = _worker_major(edge_index[0])
    dst3 = _worker_major(edge_index[1])
    zeros = jnp.zeros((CH, D), jnp.float32)

    nf = node_features
    ef = edge_features
    for i in range(edge_W1.shape[0]):
        w1 = edge_W1[i]
        ps, pd = _project_tc(nf, w1[:D], w1[D:2 * D], edge_b1[i].reshape(1, D))
        g1, g2 = gather_sc(ps, pd, src3, dst3)
        ef = _edge_tc(g1, g2, ef, w1[2 * D:],
                      edge_W2[i], edge_b2[i].reshape(1, D),
                      edge_W3[i], edge_b3[i].reshape(1, D))
        parts = scatter_sc(ef, dst3, zeros)
        nw1 = node_W1[i]
        nf = _node_tc(nf, parts[0, :N], parts[1, :N],
                      nw1[:D], nw1[D:], node_b1[i].reshape(1, D),
                      node_W2[i], node_b2[i].reshape(1, D),
                      node_W3[i], node_b3[i].reshape(1, D))
    return nf


# speedup vs baseline: 3.0816x; 1.1326x over previous
import functools

import jax
import jax.numpy as jnp
from jax import lax
from jax.experimental import pallas as pl
from jax.experimental.pallas import tpu as pltpu
from jax.experimental.pallas import tpu_sc as plsc

N = 10000
E = 160000
D = 128
NC = 2
NS = 16
NW = NC * NS
CH = 128
ROWS = E // CH
RPW = -(-ROWS // NW)
NA = 10240
NPS = NA // NS


def _worker_nrows(wid):
    return jnp.where(wid < ROWS - (RPW - 1) * NW, RPW, RPW - 1)


@functools.cache
def _sc_kernels():
    mesh = plsc.VectorSubcoreMesh(core_axis_name="c", subcore_axis_name="s",
                                  num_cores=NC, num_subcores=NS)

    @functools.partial(
        pl.kernel,
        out_type=jax.ShapeDtypeStruct((E, D), jnp.float32),
        mesh=mesh,
        scratch_types=[
            pltpu.VMEM((RPW, CH), jnp.int32),
            pltpu.VMEM((RPW, CH), jnp.int32),
            pltpu.VMEM((2, CH, D), jnp.float32),
            pltpu.VMEM((2, CH, D), jnp.float32),
            pltpu.SemaphoreType.DMA,
            pltpu.SemaphoreType.DMA,
            pltpu.SemaphoreType.DMA,
            pltpu.SemaphoreType.DMA,
            pltpu.SemaphoreType.DMA,
            pltpu.SemaphoreType.DMA,
        ],
    )
    def _gather_sc(ps_hbm, pd_hbm, src_hbm, dst_hbm, g_hbm,
                   sidx, didx, bufa, bufb, sa0, sa1, sb0, sb1, sw0, sw1):
        wid = lax.axis_index("s") * NC + lax.axis_index("c")
        nr = _worker_nrows(wid)
        pltpu.sync_copy(src_hbm.at[wid], sidx)
        pltpu.sync_copy(dst_hbm.at[wid], didx)
        sas = [sa0, sa1]
        sbs = [sb0, sb1]
        sws = [sw0, sw1]

        def gath(r, s):
            pltpu.async_copy(ps_hbm.at[sidx.at[r]], bufa.at[s], sas[s])
            pltpu.async_copy(pd_hbm.at[didx.at[r]], bufb.at[s], sbs[s])

        def wait_gath(r, s):
            pltpu.make_async_copy(ps_hbm.at[sidx.at[r]], bufa.at[s],
                                  sas[s]).wait()
            pltpu.make_async_copy(pd_hbm.at[didx.at[r]], bufb.at[s],
                                  sbs[s]).wait()

        def out_ref(r, s):
            row = r * NW + wid
            return g_hbm.at[pl.ds(row * CH, CH)]

        gath(0, 0)

        def step(r, carry):
            s = (r % 2).astype(jnp.int32)

            @pl.when(r + 1 < nr)
            def _():
                @pl.when(r >= 1)
                def _():
                    for s2 in (0, 1):
                        @pl.when(s2 != s)
                        def _():
                            pltpu.make_async_copy(bufa.at[s2],
                                                  out_ref(r - 1, s2),
                                                  sws[s2]).wait()
                for s2 in (0, 1):
                    @pl.when(s2 != s)
                    def _():
                        gath(r + 1, s2)

            for s2 in (0, 1):
                @pl.when(s2 == s)
                def _():
                    wait_gath(r, s2)

                    @plsc.parallel_loop(0, CH, 1, unroll=2)
                    def _(i):
                        for j in range(D // 16):
                            bufa[s2, i, pl.ds(j * 16, 16)] = (
                                bufa[s2, i, pl.ds(j * 16, 16)]
                                + bufb[s2, i, pl.ds(j * 16, 16)])

                    pltpu.async_copy(bufa.at[s2], out_ref(r, s2), sws[s2])
            return carry

        lax.fori_loop(0, nr, step, 0)

        def drain(r, carry):
            s = (r % 2).astype(jnp.int32)
            for s2 in (0, 1):
                @pl.when(s2 == s)
                def _():
                    pltpu.make_async_copy(bufa.at[s2], out_ref(r, s2),
                                          sws[s2]).wait()
            return carry

        lax.fori_loop(jnp.maximum(nr - 2, 0), nr, drain, 0)

    @functools.partial(
        pl.kernel,
        out_type=jax.ShapeDtypeStruct((NC, NA, D), jnp.float32),
        mesh=mesh,
        scratch_types=[
            pltpu.VMEM((RPW, CH), jnp.int32),
            pltpu.VMEM((CH, D), jnp.float32),
            pltpu.VMEM_SHARED((NA, D), jnp.float32),
        ],
    )
    def _scatter_sc(ef_hbm, dst_hbm, zeros_hbm, out_hbm, didx, buf, acc):
        cid = lax.axis_index("c")
        sid = lax.axis_index("s")
        wid = sid * NC + cid

        pltpu.sync_copy(zeros_hbm, buf)
        def zinit(k, carry):
            pltpu.sync_copy(buf, acc.at[pl.ds(sid * NPS + k * CH, CH)])
            return carry
        lax.fori_loop(0, NPS // CH, zinit, 0)
        pltpu.sync_copy(dst_hbm.at[wid], didx)
        plsc.subcore_barrier()

        def body(r, carry):
            row = r * NW + wid
            pltpu.sync_copy(ef_hbm.at[pl.ds(row * CH, CH)], buf)
            pltpu.sync_copy(buf, acc.at[didx.at[r]], add=True)
            return carry

        lax.fori_loop(0, _worker_nrows(wid), body, 0)
        plsc.subcore_barrier()

        def wout(k, carry):
            pltpu.sync_copy(acc.at[pl.ds(sid * NPS + k * CH, CH)], buf)
            pltpu.sync_copy(buf, out_hbm.at[cid, pl.ds(sid * NPS + k * CH, CH)])
            return carry
        lax.fori_loop(0, NPS // CH, wout, 0)

    return _gather_sc, _scatter_sc


_BN = 1000
_BE = 640


def _project_body(nf, w1s, w1d, b1, ps, pd):
    x = nf[...]
    ps[...] = jnp.dot(x, w1s[...], preferred_element_type=jnp.float32) + b1[...]
    pd[...] = jnp.dot(x, w1d[...], preferred_element_type=jnp.float32)


def _edge_body(g, ef, w1e, w2, b2, w3, b3, out):
    x = ef[...]
    h = g[...] + jnp.dot(x, w1e[...], preferred_element_type=jnp.float32)
    h = jnp.maximum(h, 0.0)
    h = jnp.dot(h, w2[...], preferred_element_type=jnp.float32) + b2[...]
    h = jnp.maximum(h, 0.0)
    out[...] = jnp.dot(h, w3[...], preferred_element_type=jnp.float32) + b3[...] + x


def _node_body(nf, a0, a1, w1a, w1b, b1, w2, b2, w3, b3, out):
    x = nf[...]
    agg = a0[...] + a1[...]
    h = (jnp.dot(x, w1a[...], preferred_element_type=jnp.float32)
         + jnp.dot(agg, w1b[...], preferred_element_type=jnp.float32) + b1[...])
    h = jnp.maximum(h, 0.0)
    h = jnp.dot(h, w2[...], preferred_element_type=jnp.float32) + b2[...]
    h = jnp.maximum(h, 0.0)
    out[...] = jnp.dot(h, w3[...], preferred_element_type=jnp.float32) + b3[...] + x


def _row_spec(block):
    return pl.BlockSpec((block, D), lambda b: (b, 0))


def _half_spec(block):
    return pl.BlockSpec((block, D // 2), lambda b: (b, 0))


def _w_spec():
    return pl.BlockSpec((D, D), lambda b: (0, 0))


def _b_spec():
    return pl.BlockSpec((1, D), lambda b: (0, 0))


_project_tc = pl.pallas_call(
    _project_body,
    grid=(N // _BN,),
    in_specs=[_row_spec(_BN), _w_spec(), _w_spec(), _b_spec()],
    out_specs=[_row_spec(_BN), _row_spec(_BN)],
    out_shape=[jax.ShapeDtypeStruct((N, D), jnp.float32),
               jax.ShapeDtypeStruct((N, D), jnp.float32)],
)

_edge_tc = pl.pallas_call(
    _edge_body,
    grid=(E // _BE,),
    in_specs=[_row_spec(_BE), _row_spec(_BE),
              _w_spec(), _w_spec(), _b_spec(), _w_spec(), _b_spec()],
    out_specs=_row_spec(_BE),
    out_shape=jax.ShapeDtypeStruct((E, D), jnp.float32),
)

_node_tc = pl.pallas_call(
    _node_body,
    grid=(N // _BN,),
    in_specs=[_row_spec(_BN), _row_spec(_BN), _row_spec(_BN),
              _w_spec(), _w_spec(), _b_spec(), _w_spec(), _b_spec(),
              _w_spec(), _b_spec()],
    out_specs=_row_spec(_BN),
    out_shape=jax.ShapeDtypeStruct((N, D), jnp.float32),
)


def _worker_major(idx):
    pad = NW * RPW * CH - E
    idx = jnp.concatenate([idx, jnp.zeros((pad,), idx.dtype)])
    return idx.reshape(RPW, NW, CH).transpose(1, 0, 2)


@jax.jit
def kernel(node_features, edge_features, edge_index,
           edge_W1, edge_b1, edge_W2, edge_b2, edge_W3, edge_b3,
           node_W1, node_b1, node_W2, node_b2, node_W3, node_b3):
    gather_sc, scatter_sc = _sc_kernels()
    src3 = _worker_major(edge_index[0])
    dst3 = _worker_major(edge_index[1])
    zeros = jnp.zeros((CH, D), jnp.float32)

    nf = node_features
    ef = edge_features
    for i in range(edge_W1.shape[0]):
        w1 = edge_W1[i]
        ps, pd = _project_tc(nf, w1[:D], w1[D:2 * D], edge_b1[i].reshape(1, D))
        g = gather_sc(ps, pd, src3, dst3)
        ef = _edge_tc(g, ef, w1[2 * D:],
                      edge_W2[i], edge_b2[i].reshape(1, D),
                      edge_W3[i], edge_b3[i].reshape(1, D))
        parts = scatter_sc(ef, dst3, zeros)
        nw1 = node_W1[i]
        nf = _node_tc(nf, parts[0, :N], parts[1, :N],
                      nw1[:D], nw1[D:], node_b1[i].reshape(1, D),
                      node_W2[i], node_b2[i].reshape(1, D),
                      node_W3[i], node_b3[i].reshape(1, D))
    return nf


# speedup vs baseline: 3.4801x; 1.1293x over previous
import functools

import jax
import jax.numpy as jnp
from jax import lax
from jax.experimental import pallas as pl
from jax.experimental.pallas import tpu as pltpu
from jax.experimental.pallas import tpu_sc as plsc

N = 10000
E = 160000
D = 128
NC = 2
NS = 16
NW = NC * NS
CH = 128
K_SL = 2
ES = E // K_SL
RS = ES // CH
RPW = -(-RS // NW)
NA = 10240
NPS = NA // NS


def _worker_nrows(wid):
    return jnp.where(wid < RS - (RPW - 1) * NW, RPW, RPW - 1)


@functools.cache
def _sc_kernels():
    mesh = plsc.VectorSubcoreMesh(core_axis_name="c", subcore_axis_name="s",
                                  num_cores=NC, num_subcores=NS)

    @functools.partial(
        pl.kernel,
        out_type=jax.ShapeDtypeStruct((ES, D), jnp.float32),
        mesh=mesh,
        scratch_types=[
            pltpu.VMEM((RPW, CH), jnp.int32),
            pltpu.VMEM((RPW, CH), jnp.int32),
            pltpu.VMEM((2, CH, D), jnp.float32),
            pltpu.VMEM((2, CH, D), jnp.float32),
            pltpu.SemaphoreType.DMA,
            pltpu.SemaphoreType.DMA,
            pltpu.SemaphoreType.DMA,
            pltpu.SemaphoreType.DMA,
            pltpu.SemaphoreType.DMA,
            pltpu.SemaphoreType.DMA,
        ],
    )
    def _gather_sc(ps_hbm, pd_hbm, src_hbm, dst_hbm, g_hbm,
                   sidx, didx, bufa, bufb, sa0, sa1, sb0, sb1, sw0, sw1):
        wid = lax.axis_index("s") * NC + lax.axis_index("c")
        nr = _worker_nrows(wid)
        pltpu.sync_copy(src_hbm.at[wid], sidx)
        pltpu.sync_copy(dst_hbm.at[wid], didx)
        sas = [sa0, sa1]
        sbs = [sb0, sb1]
        sws = [sw0, sw1]

        def gath(r, s):
            pltpu.async_copy(ps_hbm.at[sidx.at[r]], bufa.at[s], sas[s])
            pltpu.async_copy(pd_hbm.at[didx.at[r]], bufb.at[s], sbs[s])

        def wait_gath(r, s):
            pltpu.make_async_copy(ps_hbm.at[sidx.at[r]], bufa.at[s],
                                  sas[s]).wait()
            pltpu.make_async_copy(pd_hbm.at[didx.at[r]], bufb.at[s],
                                  sbs[s]).wait()

        def out_ref(r, s):
            row = r * NW + wid
            return g_hbm.at[pl.ds(row * CH, CH)]

        gath(0, 0)

        def step(r, carry):
            s = (r % 2).astype(jnp.int32)

            @pl.when(r + 1 < nr)
            def _():
                @pl.when(r >= 1)
                def _():
                    for s2 in (0, 1):
                        @pl.when(s2 != s)
                        def _():
                            pltpu.make_async_copy(bufa.at[s2],
                                                  out_ref(r - 1, s2),
                                                  sws[s2]).wait()
                for s2 in (0, 1):
                    @pl.when(s2 != s)
                    def _():
                        gath(r + 1, s2)

            for s2 in (0, 1):
                @pl.when(s2 == s)
                def _():
                    wait_gath(r, s2)

                    @plsc.parallel_loop(0, CH, 1, unroll=2)
                    def _(i):
                        for j in range(D // 16):
                            bufa[s2, i, pl.ds(j * 16, 16)] = (
                                bufa[s2, i, pl.ds(j * 16, 16)]
                                + bufb[s2, i, pl.ds(j * 16, 16)])

                    pltpu.async_copy(bufa.at[s2], out_ref(r, s2), sws[s2])
            return carry

        lax.fori_loop(0, nr, step, 0)

        def drain(r, carry):
            s = (r % 2).astype(jnp.int32)
            for s2 in (0, 1):
                @pl.when(s2 == s)
                def _():
                    pltpu.make_async_copy(bufa.at[s2], out_ref(r, s2),
                                          sws[s2]).wait()
            return carry

        lax.fori_loop(jnp.maximum(nr - 2, 0), nr, drain, 0)

    @functools.partial(
        pl.kernel,
        out_type=jax.ShapeDtypeStruct((NC, NA, D), jnp.float32),
        mesh=mesh,
        scratch_types=[
            pltpu.VMEM((RPW, CH), jnp.int32),
            pltpu.VMEM((CH, D), jnp.float32),
            pltpu.VMEM_SHARED((NA, D), jnp.float32),
        ],
    )
    def _scatter_sc(ef_hbm, dst_hbm, zeros_hbm, out_hbm, didx, buf, acc):
        cid = lax.axis_index("c")
        sid = lax.axis_index("s")
        wid = sid * NC + cid

        pltpu.sync_copy(zeros_hbm, buf)
        def zinit(k, carry):
            pltpu.sync_copy(buf, acc.at[pl.ds(sid * NPS + k * CH, CH)])
            return carry
        lax.fori_loop(0, NPS // CH, zinit, 0)
        pltpu.sync_copy(dst_hbm.at[wid], didx)
        plsc.subcore_barrier()

        def body(r, carry):
            row = r * NW + wid
            pltpu.sync_copy(ef_hbm.at[pl.ds(row * CH, CH)], buf)
            pltpu.sync_copy(buf, acc.at[didx.at[r]], add=True)
            return carry

        lax.fori_loop(0, _worker_nrows(wid), body, 0)
        plsc.subcore_barrier()

        def wout(k, carry):
            pltpu.sync_copy(acc.at[pl.ds(sid * NPS + k * CH, CH)], buf)
            pltpu.sync_copy(buf, out_hbm.at[cid, pl.ds(sid * NPS + k * CH, CH)])
            return carry
        lax.fori_loop(0, NPS // CH, wout, 0)

    return _gather_sc, _scatter_sc


_BN = 1000
_BE = 640


def _project_body(nf, w1s, w1d, b1, ps, pd):
    x = nf[...]
    ps[...] = jnp.dot(x, w1s[...], preferred_element_type=jnp.float32) + b1[...]
    pd[...] = jnp.dot(x, w1d[...], preferred_element_type=jnp.float32)


def _edge_body(g, ef, w1e, w2, b2, w3, b3, out):
    x = ef[...]
    h = g[...] + jnp.dot(x, w1e[...], preferred_element_type=jnp.float32)
    h = jnp.maximum(h, 0.0)
    h = jnp.dot(h, w2[...], preferred_element_type=jnp.float32) + b2[...]
    h = jnp.maximum(h, 0.0)
    out[...] = jnp.dot(h, w3[...], preferred_element_type=jnp.float32) + b3[...] + x


def _node_body(nf, a0, a1, a2, a3, w1a, w1b, b1, w2, b2, w3, b3, out):
    x = nf[...]
    agg = (a0[...] + a1[...]) + (a2[...] + a3[...])
    h = (jnp.dot(x, w1a[...], preferred_element_type=jnp.float32)
         + jnp.dot(agg, w1b[...], preferred_element_type=jnp.float32) + b1[...])
    h = jnp.maximum(h, 0.0)
    h = jnp.dot(h, w2[...], preferred_element_type=jnp.float32) + b2[...]
    h = jnp.maximum(h, 0.0)
    out[...] = jnp.dot(h, w3[...], preferred_element_type=jnp.float32) + b3[...] + x


def _row_spec(block):
    return pl.BlockSpec((block, D), lambda b: (b, 0))


def _half_spec(block):
    return pl.BlockSpec((block, D // 2), lambda b: (b, 0))


def _w_spec():
    return pl.BlockSpec((D, D), lambda b: (0, 0))


def _b_spec():
    return pl.BlockSpec((1, D), lambda b: (0, 0))


_project_tc = pl.pallas_call(
    _project_body,
    grid=(N // _BN,),
    in_specs=[_row_spec(_BN), _w_spec(), _w_spec(), _b_spec()],
    out_specs=[_row_spec(_BN), _row_spec(_BN)],
    out_shape=[jax.ShapeDtypeStruct((N, D), jnp.float32),
               jax.ShapeDtypeStruct((N, D), jnp.float32)],
)

def _make_edge_tc(off_blocks):
    return pl.pallas_call(
        _edge_body,
        grid=(ES // _BE,),
        in_specs=[_row_spec(_BE),
                  pl.BlockSpec((_BE, D), lambda b: (b + off_blocks, 0)),
                  _w_spec(), _w_spec(), _b_spec(), _w_spec(), _b_spec()],
        out_specs=_row_spec(_BE),
        out_shape=jax.ShapeDtypeStruct((ES, D), jnp.float32),
    )


_edge_tc = _make_edge_tc(0)
_edge_tc_first = [_make_edge_tc(s * (ES // _BE)) for s in range(K_SL)]

_node_tc = pl.pallas_call(
    _node_body,
    grid=(N // _BN,),
    in_specs=[_row_spec(_BN), _row_spec(_BN), _row_spec(_BN),
              _row_spec(_BN), _row_spec(_BN),
              _w_spec(), _w_spec(), _b_spec(), _w_spec(), _b_spec(),
              _w_spec(), _b_spec()],
    out_specs=_row_spec(_BN),
    out_shape=jax.ShapeDtypeStruct((N, D), jnp.float32),
)


def _worker_major(idx, s):
    sl = idx[s * ES:(s + 1) * ES]
    pad = NW * RPW * CH - ES
    sl = jnp.concatenate([sl, jnp.zeros((pad,), sl.dtype)])
    return sl.reshape(RPW, NW, CH).transpose(1, 0, 2)


@jax.jit
def kernel(node_features, edge_features, edge_index,
           edge_W1, edge_b1, edge_W2, edge_b2, edge_W3, edge_b3,
           node_W1, node_b1, node_W2, node_b2, node_W3, node_b3):
    gather_sc, scatter_sc = _sc_kernels()
    src3 = [_worker_major(edge_index[0], s) for s in range(K_SL)]
    dst3 = [_worker_major(edge_index[1], s) for s in range(K_SL)]
    zeros = jnp.zeros((CH, D), jnp.float32)

    nf = node_features
    efs = None
    for i in range(edge_W1.shape[0]):
        w1 = edge_W1[i]
        ew = (w1[2 * D:], edge_W2[i], edge_b2[i].reshape(1, D),
              edge_W3[i], edge_b3[i].reshape(1, D))
        ps, pd = _project_tc(nf, w1[:D], w1[D:2 * D], edge_b1[i].reshape(1, D))
        gs = [gather_sc(ps, pd, src3[s], dst3[s]) for s in range(K_SL)]
        if efs is None:
            efs = [_edge_tc_first[s](gs[s], edge_features, *ew)
                   for s in range(K_SL)]
        else:
            efs = [_edge_tc(gs[s], efs[s], *ew) for s in range(K_SL)]
        parts = [scatter_sc(efs[s], dst3[s], zeros) for s in range(K_SL)]
        nw1 = node_W1[i]
        nf = _node_tc(nf, parts[0][0, :N], parts[0][1, :N],
                      parts[1][0, :N], parts[1][1, :N],
                      nw1[:D], nw1[D:], node_b1[i].reshape(1, D),
                      node_W2[i], node_b2[i].reshape(1, D),
                      node_W3[i], node_b3[i].reshape(1, D))
    return nf


# speedup vs baseline: 3.6155x; 1.0389x over previous
import functools

import jax
import jax.numpy as jnp
from jax import lax
from jax.experimental import pallas as pl
from jax.experimental.pallas import tpu as pltpu
from jax.experimental.pallas import tpu_sc as plsc

N = 10000
E = 160000
D = 128
NC = 2
NS = 16
NW = NC * NS
CH = 128
K_SL = 2
ES = E // K_SL
RS = ES // CH
RPW = -(-RS // NW)
NA = 10240
NPS = NA // NS


def _worker_nrows(wid):
    return jnp.where(wid < RS - (RPW - 1) * NW, RPW, RPW - 1)


@functools.cache
def _sc_kernels():
    mesh = plsc.VectorSubcoreMesh(core_axis_name="c", subcore_axis_name="s",
                                  num_cores=NC, num_subcores=NS)

    @functools.partial(
        pl.kernel,
        out_type=jax.ShapeDtypeStruct((ES, D), jnp.float32),
        mesh=mesh,
        scratch_types=[
            pltpu.VMEM((RPW, CH), jnp.int32),
            pltpu.VMEM((RPW, CH), jnp.int32),
            pltpu.VMEM((2, CH, D), jnp.float32),
            pltpu.VMEM((2, CH, D), jnp.float32),
            pltpu.SemaphoreType.DMA,
            pltpu.SemaphoreType.DMA,
            pltpu.SemaphoreType.DMA,
            pltpu.SemaphoreType.DMA,
            pltpu.SemaphoreType.DMA,
            pltpu.SemaphoreType.DMA,
        ],
    )
    def _gather_sc(ps_hbm, pd_hbm, src_hbm, dst_hbm, g_hbm,
                   sidx, didx, bufa, bufb, sa0, sa1, sb0, sb1, sw0, sw1):
        wid = lax.axis_index("s") * NC + lax.axis_index("c")
        nr = _worker_nrows(wid)
        pltpu.sync_copy(src_hbm.at[wid], sidx)
        pltpu.sync_copy(dst_hbm.at[wid], didx)
        sas = [sa0, sa1]
        sbs = [sb0, sb1]
        sws = [sw0, sw1]

        def gath(r, s):
            pltpu.async_copy(ps_hbm.at[sidx.at[r]], bufa.at[s], sas[s])
            pltpu.async_copy(pd_hbm.at[didx.at[r]], bufb.at[s], sbs[s])

        def wait_gath(r, s):
            pltpu.make_async_copy(ps_hbm.at[sidx.at[r]], bufa.at[s],
                                  sas[s]).wait()
            pltpu.make_async_copy(pd_hbm.at[didx.at[r]], bufb.at[s],
                                  sbs[s]).wait()

        def out_ref(r, s):
            row = r * NW + wid
            return g_hbm.at[pl.ds(row * CH, CH)]

        gath(0, 0)

        def step(r, carry):
            s = (r % 2).astype(jnp.int32)

            @pl.when(r + 1 < nr)
            def _():
                @pl.when(r >= 1)
                def _():
                    for s2 in (0, 1):
                        @pl.when(s2 != s)
                        def _():
                            pltpu.make_async_copy(bufa.at[s2],
                                                  out_ref(r - 1, s2),
                                                  sws[s2]).wait()
                for s2 in (0, 1):
                    @pl.when(s2 != s)
                    def _():
                        gath(r + 1, s2)

            for s2 in (0, 1):
                @pl.when(s2 == s)
                def _():
                    wait_gath(r, s2)

                    @plsc.parallel_loop(0, CH, 1, unroll=2)
                    def _(i):
                        for j in range(D // 16):
                            bufa[s2, i, pl.ds(j * 16, 16)] = (
                                bufa[s2, i, pl.ds(j * 16, 16)]
                                + bufb[s2, i, pl.ds(j * 16, 16)])

                    pltpu.async_copy(bufa.at[s2], out_ref(r, s2), sws[s2])
            return carry

        lax.fori_loop(0, nr, step, 0)

        def drain(r, carry):
            s = (r % 2).astype(jnp.int32)
            for s2 in (0, 1):
                @pl.when(s2 == s)
                def _():
                    pltpu.make_async_copy(bufa.at[s2], out_ref(r, s2),
                                          sws[s2]).wait()
            return carry

        lax.fori_loop(jnp.maximum(nr - 2, 0), nr, drain, 0)

    @functools.partial(
        pl.kernel,
        out_type=jax.ShapeDtypeStruct((NC, NA, D), jnp.float32),
        mesh=mesh,
        scratch_types=[
            pltpu.VMEM((RPW, CH), jnp.int32),
            pltpu.VMEM((2, CH, D), jnp.float32),
            pltpu.VMEM_SHARED((NA, D), jnp.float32),
            pltpu.SemaphoreType.DMA,
            pltpu.SemaphoreType.DMA,
        ],
    )
    def _scatter_sc(ef_hbm, dst_hbm, zeros_hbm, out_hbm, didx, buf, acc,
                    sr0, sr1):
        cid = lax.axis_index("c")
        sid = lax.axis_index("s")
        wid = sid * NC + cid
        nr = _worker_nrows(wid)
        srs = [sr0, sr1]

        pltpu.sync_copy(zeros_hbm, buf.at[0])
        def zinit(k, carry):
            pltpu.sync_copy(buf.at[0], acc.at[pl.ds(sid * NPS + k * CH, CH)])
            return carry
        lax.fori_loop(0, NPS // CH, zinit, 0)
        pltpu.sync_copy(dst_hbm.at[wid], didx)
        plsc.subcore_barrier()

        def rd(r, s):
            row = r * NW + wid
            pltpu.async_copy(ef_hbm.at[pl.ds(row * CH, CH)], buf.at[s],
                             srs[s])

        def wait_rd(r, s):
            row = r * NW + wid
            pltpu.make_async_copy(ef_hbm.at[pl.ds(row * CH, CH)], buf.at[s],
                                  srs[s]).wait()

        rd(0, 0)

        def body(r, carry):
            s = (r % 2).astype(jnp.int32)
            for s2 in (0, 1):
                @pl.when((s2 == s) & (r + 1 < nr))
                def _():
                    rd(r + 1, 1 - s2)
                @pl.when(s2 == s)
                def _():
                    wait_rd(r, s2)
                    pltpu.sync_copy(buf.at[s2], acc.at[didx.at[r]], add=True)
            return carry

        lax.fori_loop(0, nr, body, 0)
        plsc.subcore_barrier()

        def wout(k, carry):
            pltpu.sync_copy(acc.at[pl.ds(sid * NPS + k * CH, CH)], buf.at[0])
            pltpu.sync_copy(buf.at[0],
                            out_hbm.at[cid, pl.ds(sid * NPS + k * CH, CH)])
            return carry
        lax.fori_loop(0, NPS // CH, wout, 0)

    return _gather_sc, _scatter_sc


_BN = 1000
_BE = 640


def _project_body(nf, w1s, w1d, b1, ps, pd):
    x = nf[...]
    ps[...] = jnp.dot(x, w1s[...], preferred_element_type=jnp.float32) + b1[...]
    pd[...] = jnp.dot(x, w1d[...], preferred_element_type=jnp.float32)


def _edge_body(g, ef, w1e, w2, b2, w3, b3, out):
    x = ef[...]
    h = g[...] + jnp.dot(x, w1e[...], preferred_element_type=jnp.float32)
    h = jnp.maximum(h, 0.0)
    h = jnp.dot(h, w2[...], preferred_element_type=jnp.float32) + b2[...]
    h = jnp.maximum(h, 0.0)
    out[...] = jnp.dot(h, w3[...], preferred_element_type=jnp.float32) + b3[...] + x


def _node_body(nf, a0, a1, a2, a3, w1a, w1b, b1, w2, b2, w3, b3, out):
    x = nf[...]
    agg = (a0[...] + a1[...]) + (a2[...] + a3[...])
    h = (jnp.dot(x, w1a[...], preferred_element_type=jnp.float32)
         + jnp.dot(agg, w1b[...], preferred_element_type=jnp.float32) + b1[...])
    h = jnp.maximum(h, 0.0)
    h = jnp.dot(h, w2[...], preferred_element_type=jnp.float32) + b2[...]
    h = jnp.maximum(h, 0.0)
    out[...] = jnp.dot(h, w3[...], preferred_element_type=jnp.float32) + b3[...] + x


def _row_spec(block):
    return pl.BlockSpec((block, D), lambda b: (b, 0))


def _half_spec(block):
    return pl.BlockSpec((block, D // 2), lambda b: (b, 0))


def _w_spec():
    return pl.BlockSpec((D, D), lambda b: (0, 0))


def _b_spec():
    return pl.BlockSpec((1, D), lambda b: (0, 0))


_project_tc = pl.pallas_call(
    _project_body,
    grid=(N // _BN,),
    in_specs=[_row_spec(_BN), _w_spec(), _w_spec(), _b_spec()],
    out_specs=[_row_spec(_BN), _row_spec(_BN)],
    out_shape=[jax.ShapeDtypeStruct((N, D), jnp.float32),
               jax.ShapeDtypeStruct((N, D), jnp.float32)],
)

def _make_edge_tc(off_blocks):
    return pl.pallas_call(
        _edge_body,
        grid=(ES // _BE,),
        in_specs=[_row_spec(_BE),
                  pl.BlockSpec((_BE, D), lambda b: (b + off_blocks, 0)),
                  _w_spec(), _w_spec(), _b_spec(), _w_spec(), _b_spec()],
        out_specs=_row_spec(_BE),
        out_shape=jax.ShapeDtypeStruct((ES, D), jnp.float32),
    )


_edge_tc = _make_edge_tc(0)
_edge_tc_first = [_make_edge_tc(s * (ES // _BE)) for s in range(K_SL)]

_node_tc = pl.pallas_call(
    _node_body,
    grid=(N // _BN,),
    in_specs=[_row_spec(_BN), _row_spec(_BN), _row_spec(_BN),
              _row_spec(_BN), _row_spec(_BN),
              _w_spec(), _w_spec(), _b_spec(), _w_spec(), _b_spec(),
              _w_spec(), _b_spec()],
    out_specs=_row_spec(_BN),
    out_shape=jax.ShapeDtypeStruct((N, D), jnp.float32),
)


def _worker_major(idx, s):
    sl = idx[s * ES:(s + 1) * ES]
    pad = NW * RPW * CH - ES
    sl = jnp.concatenate([sl, jnp.zeros((pad,), sl.dtype)])
    return sl.reshape(RPW, NW, CH).transpose(1, 0, 2)


@jax.jit
def kernel(node_features, edge_features, edge_index,
           edge_W1, edge_b1, edge_W2, edge_b2, edge_W3, edge_b3,
           node_W1, node_b1, node_W2, node_b2, node_W3, node_b3):
    gather_sc, scatter_sc = _sc_kernels()
    src3 = [_worker_major(edge_index[0], s) for s in range(K_SL)]
    dst3 = [_worker_major(edge_index[1], s) for s in range(K_SL)]
    zeros = jnp.zeros((CH, D), jnp.float32)

    nf = node_features
    efs = None
    for i in range(edge_W1.shape[0]):
        w1 = edge_W1[i]
        ew = (w1[2 * D:], edge_W2[i], edge_b2[i].reshape(1, D),
              edge_W3[i], edge_b3[i].reshape(1, D))
        ps, pd = _project_tc(nf, w1[:D], w1[D:2 * D], edge_b1[i].reshape(1, D))
        gs = [gather_sc(ps, pd, src3[s], dst3[s]) for s in range(K_SL)]
        if efs is None:
            efs = [_edge_tc_first[s](gs[s], edge_features, *ew)
                   for s in range(K_SL)]
        else:
            efs = [_edge_tc(gs[s], efs[s], *ew) for s in range(K_SL)]
        parts = [scatter_sc(efs[s], dst3[s], zeros) for s in range(K_SL)]
        nw1 = node_W1[i]
        nf = _node_tc(nf, parts[0][0, :N], parts[0][1, :N],
                      parts[1][0, :N], parts[1][1, :N],
                      nw1[:D], nw1[D:], node_b1[i].reshape(1, D),
                      node_W2[i], node_b2[i].reshape(1, D),
                      node_W3[i], node_b3[i].reshape(1, D))
    return nf


# speedup vs baseline: 3.8308x; 1.0595x over previous
import functools

import jax
import jax.numpy as jnp
from jax import lax
from jax.experimental import pallas as pl
from jax.experimental.pallas import tpu as pltpu
from jax.experimental.pallas import tpu_sc as plsc

N = 10000
E = 160000
D = 128
NC = 2
NS = 16
NW = NC * NS
CH = 128
K_SL = 2
ES = E // K_SL
RS = ES // CH
RPW = -(-RS // NW)
NA = 10240
NPS = NA // NS


def _worker_nrows(wid):
    return jnp.where(wid < RS - (RPW - 1) * NW, RPW, RPW - 1)


@functools.cache
def _sc_kernels():
    mesh = plsc.VectorSubcoreMesh(core_axis_name="c", subcore_axis_name="s",
                                  num_cores=NC, num_subcores=NS)

    @functools.partial(
        pl.kernel,
        out_type=jax.ShapeDtypeStruct((ES // 2, D), jnp.float32),
        mesh=mesh,
        scratch_types=[
            pltpu.VMEM((RPW, CH), jnp.int32),
            pltpu.VMEM((RPW, CH), jnp.int32),
            pltpu.VMEM((2, CH, D), jnp.float32),
            pltpu.VMEM((2, CH, D), jnp.float32),
            pltpu.SemaphoreType.DMA,
            pltpu.SemaphoreType.DMA,
            pltpu.SemaphoreType.DMA,
            pltpu.SemaphoreType.DMA,
            pltpu.SemaphoreType.DMA,
            pltpu.SemaphoreType.DMA,
        ],
    )
    def _gather_sc(ps_hbm, pd_hbm, src_hbm, dst_hbm, g_hbm,
                   sidx, didx, bufa, bufb, sa0, sa1, sb0, sb1, sw0, sw1):
        wid = lax.axis_index("s") * NC + lax.axis_index("c")
        nr = _worker_nrows(wid)
        pltpu.sync_copy(src_hbm.at[wid], sidx)
        pltpu.sync_copy(dst_hbm.at[wid], didx)
        sas = [sa0, sa1]
        sbs = [sb0, sb1]
        sws = [sw0, sw1]
        HC = CH // 2

        def gath(r, s):
            pltpu.async_copy(ps_hbm.at[sidx.at[r]], bufa.at[s], sas[s])
            pltpu.async_copy(pd_hbm.at[didx.at[r]], bufb.at[s], sbs[s])

        def wait_gath(r, s):
            pltpu.make_async_copy(ps_hbm.at[sidx.at[r]], bufa.at[s],
                                  sas[s]).wait()
            pltpu.make_async_copy(pd_hbm.at[didx.at[r]], bufb.at[s],
                                  sbs[s]).wait()

        def out_ref(r, s):
            row = r * NW + wid
            return g_hbm.at[pl.ds(row * HC, HC)]

        def wait_w(r, s):
            pltpu.make_async_copy(bufb.at[s, pl.ds(0, HC)], out_ref(r, s),
                                  sws[s]).wait()

        c7fff = jnp.full((16,), 0x7FFF, jnp.int32)
        c16 = jnp.full((16,), 16, jnp.int32)
        c1 = jnp.full((16,), 1, jnp.int32)
        cmask = jnp.full((16,), -65536, jnp.int32)

        def rtne16(bits):
            return bits + c7fff + (lax.shift_right_logical(bits, c16) & c1)

        gath(0, 0)

        def step(r, carry):
            s = (r % 2).astype(jnp.int32)

            @pl.when(r + 1 < nr)
            def _():
                @pl.when(r >= 1)
                def _():
                    for s2 in (0, 1):
                        @pl.when(s2 != s)
                        def _():
                            wait_w(r - 1, s2)
                for s2 in (0, 1):
                    @pl.when(s2 != s)
                    def _():
                        gath(r + 1, s2)

            for s2 in (0, 1):
                @pl.when(s2 == s)
                def _():
                    wait_gath(r, s2)

                    @plsc.parallel_loop(0, HC, 1, unroll=2)
                    def _(i):
                        for j in range(D // 16):
                            c = pl.ds(j * 16, 16)
                            lo = (bufa[s2, i, c] + bufb[s2, i, c])
                            hi = (bufa[s2, i + HC, c] + bufb[s2, i + HC, c])
                            lo_u = rtne16(lax.bitcast_convert_type(lo, jnp.int32))
                            hi_u = rtne16(lax.bitcast_convert_type(hi, jnp.int32))
                            packed = (
                                lax.shift_right_logical(lo_u, c16)
                                | (hi_u & cmask))
                            bufb[s2, i, c] = lax.bitcast_convert_type(
                                packed, jnp.float32)

                    pltpu.async_copy(bufb.at[s2, pl.ds(0, HC)],
                                     out_ref(r, s2), sws[s2])
            return carry

        lax.fori_loop(0, nr, step, 0)

        def drain(r, carry):
            s = (r % 2).astype(jnp.int32)
            for s2 in (0, 1):
                @pl.when(s2 == s)
                def _():
                    wait_w(r, s2)
            return carry

        lax.fori_loop(jnp.maximum(nr - 2, 0), nr, drain, 0)

    @functools.partial(
        pl.kernel,
        out_type=jax.ShapeDtypeStruct((NC, NA, D), jnp.float32),
        mesh=mesh,
        scratch_types=[
            pltpu.VMEM((RPW, CH), jnp.int32),
            pltpu.VMEM((2, CH, D), jnp.float32),
            pltpu.VMEM_SHARED((NA, D), jnp.float32),
            pltpu.SemaphoreType.DMA,
            pltpu.SemaphoreType.DMA,
        ],
    )
    def _scatter_sc(ef_hbm, dst_hbm, zeros_hbm, out_hbm, didx, buf, acc,
                    sr0, sr1):
        cid = lax.axis_index("c")
        sid = lax.axis_index("s")
        wid = sid * NC + cid
        nr = _worker_nrows(wid)
        srs = [sr0, sr1]

        pltpu.sync_copy(zeros_hbm, buf.at[0])
        def zinit(k, carry):
            pltpu.sync_copy(buf.at[0], acc.at[pl.ds(sid * NPS + k * CH, CH)])
            return carry
        lax.fori_loop(0, NPS // CH, zinit, 0)
        pltpu.sync_copy(dst_hbm.at[wid], didx)
        plsc.subcore_barrier()

        def rd(r, s):
            row = r * NW + wid
            pltpu.async_copy(ef_hbm.at[pl.ds(row * CH, CH)], buf.at[s],
                             srs[s])

        def wait_rd(r, s):
            row = r * NW + wid
            pltpu.make_async_copy(ef_hbm.at[pl.ds(row * CH, CH)], buf.at[s],
                                  srs[s]).wait()

        rd(0, 0)

        def body(r, carry):
            s = (r % 2).astype(jnp.int32)
            for s2 in (0, 1):
                @pl.when((s2 == s) & (r + 1 < nr))
                def _():
                    rd(r + 1, 1 - s2)
                @pl.when(s2 == s)
                def _():
                    wait_rd(r, s2)
                    pltpu.sync_copy(buf.at[s2], acc.at[didx.at[r]], add=True)
            return carry

        lax.fori_loop(0, nr, body, 0)
        plsc.subcore_barrier()

        def wout(k, carry):
            pltpu.sync_copy(acc.at[pl.ds(sid * NPS + k * CH, CH)], buf.at[0])
            pltpu.sync_copy(buf.at[0],
                            out_hbm.at[cid, pl.ds(sid * NPS + k * CH, CH)])
            return carry
        lax.fori_loop(0, NPS // CH, wout, 0)

    return _gather_sc, _scatter_sc


_BN = 1000
_BE = 640


def _project_body(nf, w1s, w1d, b1, ps, pd):
    x = nf[...]
    ps[...] = jnp.dot(x, w1s[...], preferred_element_type=jnp.float32) + b1[...]
    pd[...] = jnp.dot(x, w1d[...], preferred_element_type=jnp.float32)


def _edge_body(gp, ef, w1e, w2, b2, w3, b3, out):
    x = ef[...]
    w = lax.bitcast_convert_type(gp[...], jnp.uint32)
    lo = lax.bitcast_convert_type(
        lax.shift_left(w, jnp.uint32(16)), jnp.float32)
    hi = lax.bitcast_convert_type(w & jnp.uint32(0xFFFF0000), jnp.float32)
    nch = _BE // CH
    g = jnp.concatenate(
        [lo.reshape(nch, CH // 2, D), hi.reshape(nch, CH // 2, D)],
        axis=1).reshape(_BE, D)
    h = g + jnp.dot(x, w1e[...], preferred_element_type=jnp.float32)
    h = jnp.maximum(h, 0.0)
    h = jnp.dot(h, w2[...], preferred_element_type=jnp.float32) + b2[...]
    h = jnp.maximum(h, 0.0)
    out[...] = jnp.dot(h, w3[...], preferred_element_type=jnp.float32) + b3[...] + x


def _node_body(nf, a0, a1, a2, a3, w1a, w1b, b1, w2, b2, w3, b3, out):
    x = nf[...]
    agg = (a0[...] + a1[...]) + (a2[...] + a3[...])
    h = (jnp.dot(x, w1a[...], preferred_element_type=jnp.float32)
         + jnp.dot(agg, w1b[...], preferred_element_type=jnp.float32) + b1[...])
    h = jnp.maximum(h, 0.0)
    h = jnp.dot(h, w2[...], preferred_element_type=jnp.float32) + b2[...]
    h = jnp.maximum(h, 0.0)
    out[...] = jnp.dot(h, w3[...], preferred_element_type=jnp.float32) + b3[...] + x


def _row_spec(block):
    return pl.BlockSpec((block, D), lambda b: (b, 0))


def _half_spec(block):
    return pl.BlockSpec((block, D // 2), lambda b: (b, 0))


def _w_spec():
    return pl.BlockSpec((D, D), lambda b: (0, 0))


def _b_spec():
    return pl.BlockSpec((1, D), lambda b: (0, 0))


_project_tc = pl.pallas_call(
    _project_body,
    grid=(N // _BN,),
    in_specs=[_row_spec(_BN), _w_spec(), _w_spec(), _b_spec()],
    out_specs=[_row_spec(_BN), _row_spec(_BN)],
    out_shape=[jax.ShapeDtypeStruct((N, D), jnp.float32),
               jax.ShapeDtypeStruct((N, D), jnp.float32)],
)

def _make_edge_tc(off_blocks):
    return pl.pallas_call(
        _edge_body,
        grid=(ES // _BE,),
        in_specs=[_row_spec(_BE // 2),
                  pl.BlockSpec((_BE, D), lambda b: (b + off_blocks, 0)),
                  _w_spec(), _w_spec(), _b_spec(), _w_spec(), _b_spec()],
        out_specs=_row_spec(_BE),
        out_shape=jax.ShapeDtypeStruct((ES, D), jnp.float32),
    )


_edge_tc = _make_edge_tc(0)
_edge_tc_first = [_make_edge_tc(s * (ES // _BE)) for s in range(K_SL)]

_node_tc = pl.pallas_call(
    _node_body,
    grid=(N // _BN,),
    in_specs=[_row_spec(_BN), _row_spec(_BN), _row_spec(_BN),
              _row_spec(_BN), _row_spec(_BN),
              _w_spec(), _w_spec(), _b_spec(), _w_spec(), _b_spec(),
              _w_spec(), _b_spec()],
    out_specs=_row_spec(_BN),
    out_shape=jax.ShapeDtypeStruct((N, D), jnp.float32),
)


def _worker_major(idx, s):
    sl = idx[s * ES:(s + 1) * ES]
    pad = NW * RPW * CH - ES
    sl = jnp.concatenate([sl, jnp.zeros((pad,), sl.dtype)])
    return sl.reshape(RPW, NW, CH).transpose(1, 0, 2)


@jax.jit
def kernel(node_features, edge_features, edge_index,
           edge_W1, edge_b1, edge_W2, edge_b2, edge_W3, edge_b3,
           node_W1, node_b1, node_W2, node_b2, node_W3, node_b3):
    gather_sc, scatter_sc = _sc_kernels()
    src3 = [_worker_major(edge_index[0], s) for s in range(K_SL)]
    dst3 = [_worker_major(edge_index[1], s) for s in range(K_SL)]
    zeros = jnp.zeros((CH, D), jnp.float32)

    nf = node_features
    efs = None
    for i in range(edge_W1.shape[0]):
        w1 = edge_W1[i]
        ew = (w1[2 * D:], edge_W2[i], edge_b2[i].reshape(1, D),
              edge_W3[i], edge_b3[i].reshape(1, D))
        ps, pd = _project_tc(nf, w1[:D], w1[D:2 * D], edge_b1[i].reshape(1, D))
        gs = [gather_sc(ps, pd, src3[s], dst3[s]) for s in range(K_SL)]
        if efs is None:
            efs = [_edge_tc_first[s](gs[s], edge_features, *ew)
                   for s in range(K_SL)]
        else:
            efs = [_edge_tc(gs[s], efs[s], *ew) for s in range(K_SL)]
        parts = [scatter_sc(efs[s], dst3[s], zeros) for s in range(K_SL)]
        nw1 = node_W1[i]
        nf = _node_tc(nf, parts[0][0, :N], parts[0][1, :N],
                      parts[1][0, :N], parts[1][1, :N],
                      nw1[:D], nw1[D:], node_b1[i].reshape(1, D),
                      node_W2[i], node_b2[i].reshape(1, D),
                      node_W3[i], node_b3[i].reshape(1, D))
    return nf


# speedup vs baseline: 3.9670x; 1.0356x over previous
import functools

import jax
import jax.numpy as jnp
from jax import lax
from jax.experimental import pallas as pl
from jax.experimental.pallas import tpu as pltpu
from jax.experimental.pallas import tpu_sc as plsc

N = 10000
E = 160000
D = 128
NC = 2
NS = 16
NW = NC * NS
CH = 128
K_SL = 2
ES = E // K_SL
RS = ES // CH
RPW = -(-RS // NW)
NA = 10240
NPS = NA // NS


def _worker_nrows(wid):
    return jnp.where(wid < RS - (RPW - 1) * NW, RPW, RPW - 1)


@functools.cache
def _sc_kernels():
    mesh = plsc.VectorSubcoreMesh(core_axis_name="c", subcore_axis_name="s",
                                  num_cores=NC, num_subcores=NS)

    @functools.partial(
        pl.kernel,
        out_type=jax.ShapeDtypeStruct((ES // 2, D), jnp.float32),
        mesh=mesh,
        scratch_types=[
            pltpu.VMEM((RPW, CH), jnp.int32),
            pltpu.VMEM((RPW, CH), jnp.int32),
            pltpu.VMEM((2, CH, D), jnp.float32),
            pltpu.VMEM((2, CH, D), jnp.float32),
            pltpu.SemaphoreType.DMA,
            pltpu.SemaphoreType.DMA,
            pltpu.SemaphoreType.DMA,
            pltpu.SemaphoreType.DMA,
            pltpu.SemaphoreType.DMA,
            pltpu.SemaphoreType.DMA,
        ],
    )
    def _gather_sc(ps_hbm, pd_hbm, src_hbm, dst_hbm, g_hbm,
                   sidx, didx, bufa, bufb, sa0, sa1, sb0, sb1, sw0, sw1):
        wid = lax.axis_index("s") * NC + lax.axis_index("c")
        nr = _worker_nrows(wid)
        pltpu.sync_copy(src_hbm.at[wid], sidx)
        pltpu.sync_copy(dst_hbm.at[wid], didx)
        sas = [sa0, sa1]
        sbs = [sb0, sb1]
        sws = [sw0, sw1]
        HC = CH // 2

        def gath(r, s):
            pltpu.async_copy(ps_hbm.at[sidx.at[r]], bufa.at[s], sas[s])
            pltpu.async_copy(pd_hbm.at[didx.at[r]], bufb.at[s], sbs[s])

        def wait_gath(r, s):
            pltpu.make_async_copy(ps_hbm.at[sidx.at[r]], bufa.at[s],
                                  sas[s]).wait()
            pltpu.make_async_copy(pd_hbm.at[didx.at[r]], bufb.at[s],
                                  sbs[s]).wait()

        def out_ref(r, s):
            row = r * NW + wid
            return g_hbm.at[pl.ds(row * HC, HC)]

        def wait_w(r, s):
            pltpu.make_async_copy(bufb.at[s, pl.ds(0, HC)], out_ref(r, s),
                                  sws[s]).wait()

        c7fff = jnp.full((16,), 0x7FFF, jnp.int32)
        c16 = jnp.full((16,), 16, jnp.int32)
        c1 = jnp.full((16,), 1, jnp.int32)
        cmask = jnp.full((16,), -65536, jnp.int32)

        def rtne16(bits):
            return bits + c7fff + (lax.shift_right_logical(bits, c16) & c1)

        gath(0, 0)

        def step(r, carry):
            s = (r % 2).astype(jnp.int32)

            @pl.when(r + 1 < nr)
            def _():
                @pl.when(r >= 1)
                def _():
                    for s2 in (0, 1):
                        @pl.when(s2 != s)
                        def _():
                            wait_w(r - 1, s2)
                for s2 in (0, 1):
                    @pl.when(s2 != s)
                    def _():
                        gath(r + 1, s2)

            for s2 in (0, 1):
                @pl.when(s2 == s)
                def _():
                    wait_gath(r, s2)

                    @plsc.parallel_loop(0, HC, 1, unroll=2)
                    def _(i):
                        for j in range(D // 16):
                            c = pl.ds(j * 16, 16)
                            lo = (bufa[s2, i, c] + bufb[s2, i, c])
                            hi = (bufa[s2, i + HC, c] + bufb[s2, i + HC, c])
                            lo_u = rtne16(lax.bitcast_convert_type(lo, jnp.int32))
                            hi_u = rtne16(lax.bitcast_convert_type(hi, jnp.int32))
                            packed = (
                                lax.shift_right_logical(lo_u, c16)
                                | (hi_u & cmask))
                            bufb[s2, i, c] = lax.bitcast_convert_type(
                                packed, jnp.float32)

                    pltpu.async_copy(bufb.at[s2, pl.ds(0, HC)],
                                     out_ref(r, s2), sws[s2])
            return carry

        lax.fori_loop(0, nr, step, 0)

        def drain(r, carry):
            s = (r % 2).astype(jnp.int32)
            for s2 in (0, 1):
                @pl.when(s2 == s)
                def _():
                    wait_w(r, s2)
            return carry

        lax.fori_loop(jnp.maximum(nr - 2, 0), nr, drain, 0)

    @functools.partial(
        pl.kernel,
        out_type=jax.ShapeDtypeStruct((NC, NA, D), jnp.float32),
        mesh=mesh,
        scratch_types=[
            pltpu.VMEM((RPW, CH), jnp.int32),
            pltpu.VMEM((2, CH // 2, D), jnp.float32),
            pltpu.VMEM((CH, D), jnp.float32),
            pltpu.VMEM_SHARED((NA, D), jnp.float32),
            pltpu.SemaphoreType.DMA,
            pltpu.SemaphoreType.DMA,
        ],
    )
    def _scatter_sc(efp_hbm, dst_hbm, zeros_hbm, out_hbm, didx, bufp, bufu,
                    acc, sr0, sr1):
        cid = lax.axis_index("c")
        sid = lax.axis_index("s")
        wid = sid * NC + cid
        nr = _worker_nrows(wid)
        srs = [sr0, sr1]
        HC = CH // 2
        c16 = jnp.full((16,), 16, jnp.int32)
        cmask = jnp.full((16,), -65536, jnp.int32)

        pltpu.sync_copy(zeros_hbm, bufu)
        def zinit(k, carry):
            pltpu.sync_copy(bufu, acc.at[pl.ds(sid * NPS + k * CH, CH)])
            return carry
        lax.fori_loop(0, NPS // CH, zinit, 0)
        pltpu.sync_copy(dst_hbm.at[wid], didx)
        plsc.subcore_barrier()

        def rd(r, s):
            row = r * NW + wid
            pltpu.async_copy(efp_hbm.at[pl.ds(row * HC, HC)], bufp.at[s],
                             srs[s])

        def wait_rd(r, s):
            row = r * NW + wid
            pltpu.make_async_copy(efp_hbm.at[pl.ds(row * HC, HC)],
                                  bufp.at[s], srs[s]).wait()

        rd(0, 0)

        def body(r, carry):
            s = (r % 2).astype(jnp.int32)
            for s2 in (0, 1):
                @pl.when((s2 == s) & (r + 1 < nr))
                def _():
                    rd(r + 1, 1 - s2)
                @pl.when(s2 == s)
                def _():
                    wait_rd(r, s2)

                    @plsc.parallel_loop(0, HC, 1, unroll=2)
                    def _(i):
                        for j in range(D // 16):
                            c = pl.ds(j * 16, 16)
                            w = lax.bitcast_convert_type(bufp[s2, i, c],
                                                         jnp.int32)
                            bufu[i, c] = lax.bitcast_convert_type(
                                lax.shift_left(w, c16), jnp.float32)
                            bufu[i + HC, c] = lax.bitcast_convert_type(
                                w & cmask, jnp.float32)

                    pltpu.sync_copy(bufu, acc.at[didx.at[r]], add=True)
            return carry

        lax.fori_loop(0, nr, body, 0)
        plsc.subcore_barrier()

        def wout(k, carry):
            pltpu.sync_copy(acc.at[pl.ds(sid * NPS + k * CH, CH)], bufu)
            pltpu.sync_copy(bufu,
                            out_hbm.at[cid, pl.ds(sid * NPS + k * CH, CH)])
            return carry
        lax.fori_loop(0, NPS // CH, wout, 0)

    return _gather_sc, _scatter_sc


_BN = 1000
_BE = 640


def _project_body(nf, w1s, w1d, b1, ps, pd):
    x = nf[...]
    ps[...] = jnp.dot(x, w1s[...], preferred_element_type=jnp.float32) + b1[...]
    pd[...] = jnp.dot(x, w1d[...], preferred_element_type=jnp.float32)


def _unpack_pairs(p, nch):
    w = lax.bitcast_convert_type(p, jnp.uint32)
    lo = lax.bitcast_convert_type(
        lax.shift_left(w, jnp.uint32(16)), jnp.float32)
    hi = lax.bitcast_convert_type(w & jnp.uint32(0xFFFF0000), jnp.float32)
    return jnp.concatenate(
        [lo.reshape(nch, CH // 2, D), hi.reshape(nch, CH // 2, D)],
        axis=1).reshape(nch * CH, D)


def _pack_pairs(x, nch):
    x3 = x.reshape(nch, 2, CH // 2, D)
    def rtne(b):
        return b + jnp.uint32(0x7FFF) + (
            lax.shift_right_logical(b, jnp.uint32(16)) & jnp.uint32(1))
    lo = rtne(lax.bitcast_convert_type(x3[:, 0].reshape(nch * CH // 2, D),
                                       jnp.uint32))
    hi = rtne(lax.bitcast_convert_type(x3[:, 1].reshape(nch * CH // 2, D),
                                       jnp.uint32))
    packed = (lax.shift_right_logical(lo, jnp.uint32(16))
              | (hi & jnp.uint32(0xFFFF0000)))
    return lax.bitcast_convert_type(packed, jnp.float32)


def _edge_mlp(g, x, w1e, w2, b2, w3, b3):
    h = g + jnp.dot(x, w1e[...], preferred_element_type=jnp.float32)
    h = jnp.maximum(h, 0.0)
    h = jnp.dot(h, w2[...], preferred_element_type=jnp.float32) + b2[...]
    h = jnp.maximum(h, 0.0)
    return jnp.dot(h, w3[...], preferred_element_type=jnp.float32) + b3[...] + x


def _edge_body_first(gp, ef, w1e, w2, b2, w3, b3, out):
    nch = _BE // CH
    g = _unpack_pairs(gp[...], nch)
    out[...] = _pack_pairs(_edge_mlp(g, ef[...], w1e, w2, b2, w3, b3), nch)


def _edge_body(gp, efp, w1e, w2, b2, w3, b3, out):
    nch = _BE // CH
    g = _unpack_pairs(gp[...], nch)
    x = _unpack_pairs(efp[...], nch)
    out[...] = _pack_pairs(_edge_mlp(g, x, w1e, w2, b2, w3, b3), nch)


def _node_body(nf, a0, a1, a2, a3, w1a, w1b, b1, w2, b2, w3, b3, out):
    x = nf[...]
    agg = (a0[...] + a1[...]) + (a2[...] + a3[...])
    h = (jnp.dot(x, w1a[...], preferred_element_type=jnp.float32)
         + jnp.dot(agg, w1b[...], preferred_element_type=jnp.float32) + b1[...])
    h = jnp.maximum(h, 0.0)
    h = jnp.dot(h, w2[...], preferred_element_type=jnp.float32) + b2[...]
    h = jnp.maximum(h, 0.0)
    out[...] = jnp.dot(h, w3[...], preferred_element_type=jnp.float32) + b3[...] + x


def _row_spec(block):
    return pl.BlockSpec((block, D), lambda b: (b, 0))


def _half_spec(block):
    return pl.BlockSpec((block, D // 2), lambda b: (b, 0))


def _w_spec():
    return pl.BlockSpec((D, D), lambda b: (0, 0))


def _b_spec():
    return pl.BlockSpec((1, D), lambda b: (0, 0))


_project_tc = pl.pallas_call(
    _project_body,
    grid=(N // _BN,),
    in_specs=[_row_spec(_BN), _w_spec(), _w_spec(), _b_spec()],
    out_specs=[_row_spec(_BN), _row_spec(_BN)],
    out_shape=[jax.ShapeDtypeStruct((N, D), jnp.float32),
               jax.ShapeDtypeStruct((N, D), jnp.float32)],
)

def _make_edge_tc_first(off_blocks):
    return pl.pallas_call(
        _edge_body_first,
        grid=(ES // _BE,),
        in_specs=[_row_spec(_BE // 2),
                  pl.BlockSpec((_BE, D), lambda b: (b + off_blocks, 0)),
                  _w_spec(), _w_spec(), _b_spec(), _w_spec(), _b_spec()],
        out_specs=_row_spec(_BE // 2),
        out_shape=jax.ShapeDtypeStruct((ES // 2, D), jnp.float32),
    )


_edge_tc = pl.pallas_call(
    _edge_body,
    grid=(ES // _BE,),
    in_specs=[_row_spec(_BE // 2), _row_spec(_BE // 2),
              _w_spec(), _w_spec(), _b_spec(), _w_spec(), _b_spec()],
    out_specs=_row_spec(_BE // 2),
    out_shape=jax.ShapeDtypeStruct((ES // 2, D), jnp.float32),
)
_edge_tc_first = [_make_edge_tc_first(s * (ES // _BE)) for s in range(K_SL)]

_node_tc = pl.pallas_call(
    _node_body,
    grid=(N // _BN,),
    in_specs=[_row_spec(_BN), _row_spec(_BN), _row_spec(_BN),
              _row_spec(_BN), _row_spec(_BN),
              _w_spec(), _w_spec(), _b_spec(), _w_spec(), _b_spec(),
              _w_spec(), _b_spec()],
    out_specs=_row_spec(_BN),
    out_shape=jax.ShapeDtypeStruct((N, D), jnp.float32),
)


def _worker_major(idx, s):
    sl = idx[s * ES:(s + 1) * ES]
    pad = NW * RPW * CH - ES
    sl = jnp.concatenate([sl, jnp.zeros((pad,), sl.dtype)])
    return sl.reshape(RPW, NW, CH).transpose(1, 0, 2)


@jax.jit
def kernel(node_features, edge_features, edge_index,
           edge_W1, edge_b1, edge_W2, edge_b2, edge_W3, edge_b3,
           node_W1, node_b1, node_W2, node_b2, node_W3, node_b3):
    gather_sc, scatter_sc = _sc_kernels()
    src3 = [_worker_major(edge_index[0], s) for s in range(K_SL)]
    dst3 = [_worker_major(edge_index[1], s) for s in range(K_SL)]
    zeros = jnp.zeros((CH, D), jnp.float32)

    nf = node_features
    efs = None
    for i in range(edge_W1.shape[0]):
        w1 = edge_W1[i]
        ew = (w1[2 * D:], edge_W2[i], edge_b2[i].reshape(1, D),
              edge_W3[i], edge_b3[i].reshape(1, D))
        ps, pd = _project_tc(nf, w1[:D], w1[D:2 * D], edge_b1[i].reshape(1, D))
        gs = [gather_sc(ps, pd, src3[s], dst3[s]) for s in range(K_SL)]
        if efs is None:
            efs = [_edge_tc_first[s](gs[s], edge_features, *ew)
                   for s in range(K_SL)]
        else:
            efs = [_edge_tc(gs[s], efs[s], *ew) for s in range(K_SL)]
        parts = [scatter_sc(efs[s], dst3[s], zeros) for s in range(K_SL)]
        nw1 = node_W1[i]
        nf = _node_tc(nf, parts[0][0, :N], parts[0][1, :N],
                      parts[1][0, :N], parts[1][1, :N],
                      nw1[:D], nw1[D:], node_b1[i].reshape(1, D),
                      node_W2[i], node_b2[i].reshape(1, D),
                      node_W3[i], node_b3[i].reshape(1, D))
    return nf


# speedup vs baseline: 4.1403x; 1.0437x over previous
import functools

import jax
import jax.numpy as jnp
from jax import lax
from jax.experimental import pallas as pl
from jax.experimental.pallas import tpu as pltpu
from jax.experimental.pallas import tpu_sc as plsc

N = 10000
E = 160000
D = 128
NC = 2
NS = 16
NW = NC * NS
CH = 128
K_SL = 2
ES = E // K_SL
RS = ES // CH
RPW = -(-RS // NW)
NA = 10240
NPS = NA // NS


def _worker_nrows(wid):
    return jnp.where(wid < RS - (RPW - 1) * NW, RPW, RPW - 1)


@functools.cache
def _sc_kernels():
    mesh = plsc.VectorSubcoreMesh(core_axis_name="c", subcore_axis_name="s",
                                  num_cores=NC, num_subcores=NS)

    @functools.partial(
        pl.kernel,
        out_type=jax.ShapeDtypeStruct((ES // 2, D), jnp.float32),
        mesh=mesh,
        scratch_types=[
            pltpu.VMEM((RPW, CH), jnp.int32),
            pltpu.VMEM((RPW, CH), jnp.int32),
            pltpu.VMEM((2, CH, D), jnp.float32),
            pltpu.VMEM((2, CH, D), jnp.float32),
            pltpu.SemaphoreType.DMA,
            pltpu.SemaphoreType.DMA,
            pltpu.SemaphoreType.DMA,
            pltpu.SemaphoreType.DMA,
            pltpu.SemaphoreType.DMA,
            pltpu.SemaphoreType.DMA,
        ],
    )
    def _gather_sc(ps_hbm, pd_hbm, src_hbm, dst_hbm, g_hbm,
                   sidx, didx, bufa, bufb, sa0, sa1, sb0, sb1, sw0, sw1):
        wid = lax.axis_index("s") * NC + lax.axis_index("c")
        nr = _worker_nrows(wid)
        pltpu.sync_copy(src_hbm.at[wid], sidx)
        pltpu.sync_copy(dst_hbm.at[wid], didx)
        sas = [sa0, sa1]
        sbs = [sb0, sb1]
        sws = [sw0, sw1]
        HC = CH // 2

        def gath(r, s):
            pltpu.async_copy(ps_hbm.at[sidx.at[r]], bufa.at[s], sas[s])
            pltpu.async_copy(pd_hbm.at[didx.at[r]], bufb.at[s], sbs[s])

        def wait_gath(r, s):
            pltpu.make_async_copy(ps_hbm.at[sidx.at[r]], bufa.at[s],
                                  sas[s]).wait()
            pltpu.make_async_copy(pd_hbm.at[didx.at[r]], bufb.at[s],
                                  sbs[s]).wait()

        def out_ref(r, s):
            row = r * NW + wid
            return g_hbm.at[pl.ds(row * HC, HC)]

        def wait_w(r, s):
            pltpu.make_async_copy(bufb.at[s, pl.ds(0, HC)], out_ref(r, s),
                                  sws[s]).wait()

        c7fff = jnp.full((16,), 0x7FFF, jnp.int32)
        c16 = jnp.full((16,), 16, jnp.int32)
        c1 = jnp.full((16,), 1, jnp.int32)
        cmask = jnp.full((16,), -65536, jnp.int32)

        def rtne16(bits):
            return bits + c7fff + (lax.shift_right_logical(bits, c16) & c1)

        gath(0, 0)

        def step(r, carry):
            s = (r % 2).astype(jnp.int32)

            @pl.when(r + 1 < nr)
            def _():
                @pl.when(r >= 1)
                def _():
                    for s2 in (0, 1):
                        @pl.when(s2 != s)
                        def _():
                            wait_w(r - 1, s2)
                for s2 in (0, 1):
                    @pl.when(s2 != s)
                    def _():
                        gath(r + 1, s2)

            for s2 in (0, 1):
                @pl.when(s2 == s)
                def _():
                    wait_gath(r, s2)

                    @plsc.parallel_loop(0, HC, 1, unroll=2)
                    def _(i):
                        for j in range(D // 16):
                            c = pl.ds(j * 16, 16)
                            lo = (bufa[s2, i, c] + bufb[s2, i, c])
                            hi = (bufa[s2, i + HC, c] + bufb[s2, i + HC, c])
                            lo_u = rtne16(lax.bitcast_convert_type(lo, jnp.int32))
                            hi_u = rtne16(lax.bitcast_convert_type(hi, jnp.int32))
                            packed = (
                                lax.shift_right_logical(lo_u, c16)
                                | (hi_u & cmask))
                            bufb[s2, i, c] = lax.bitcast_convert_type(
                                packed, jnp.float32)

                    pltpu.async_copy(bufb.at[s2, pl.ds(0, HC)],
                                     out_ref(r, s2), sws[s2])
            return carry

        lax.fori_loop(0, nr, step, 0)

        def drain(r, carry):
            s = (r % 2).astype(jnp.int32)
            for s2 in (0, 1):
                @pl.when(s2 == s)
                def _():
                    wait_w(r, s2)
            return carry

        lax.fori_loop(jnp.maximum(nr - 2, 0), nr, drain, 0)

    @functools.partial(
        pl.kernel,
        out_type=jax.ShapeDtypeStruct((NC, NA, D), jnp.float32),
        mesh=mesh,
        scratch_types=[
            pltpu.VMEM((RPW, CH), jnp.int32),
            pltpu.VMEM((2, CH // 2, D), jnp.float32),
            pltpu.VMEM((CH, D), jnp.float32),
            pltpu.VMEM_SHARED((NA, D), jnp.float32),
            pltpu.SemaphoreType.DMA,
            pltpu.SemaphoreType.DMA,
        ],
    )
    def _scatter_sc(efp_hbm, dst_hbm, zeros_hbm, out_hbm, didx, bufp, bufu,
                    acc, sr0, sr1):
        cid = lax.axis_index("c")
        sid = lax.axis_index("s")
        wid = sid * NC + cid
        nr = _worker_nrows(wid)
        srs = [sr0, sr1]
        HC = CH // 2
        c16 = jnp.full((16,), 16, jnp.int32)
        cmask = jnp.full((16,), -65536, jnp.int32)

        pltpu.sync_copy(zeros_hbm, bufu)
        def zinit(k, carry):
            pltpu.sync_copy(bufu, acc.at[pl.ds(sid * NPS + k * CH, CH)])
            return carry
        lax.fori_loop(0, NPS // CH, zinit, 0)
        pltpu.sync_copy(dst_hbm.at[wid], didx)
        plsc.subcore_barrier()

        def rd(r, s):
            row = r * NW + wid
            pltpu.async_copy(efp_hbm.at[pl.ds(row * HC, HC)], bufp.at[s],
                             srs[s])

        def wait_rd(r, s):
            row = r * NW + wid
            pltpu.make_async_copy(efp_hbm.at[pl.ds(row * HC, HC)],
                                  bufp.at[s], srs[s]).wait()

        rd(0, 0)

        def body(r, carry):
            s = (r % 2).astype(jnp.int32)
            for s2 in (0, 1):
                @pl.when((s2 == s) & (r + 1 < nr))
                def _():
                    rd(r + 1, 1 - s2)
                @pl.when(s2 == s)
                def _():
                    wait_rd(r, s2)

                    @plsc.parallel_loop(0, HC, 1, unroll=2)
                    def _(i):
                        for j in range(D // 16):
                            c = pl.ds(j * 16, 16)
                            w = lax.bitcast_convert_type(bufp[s2, i, c],
                                                         jnp.int32)
                            bufu[i, c] = lax.bitcast_convert_type(
                                lax.shift_left(w, c16), jnp.float32)
                            bufu[i + HC, c] = lax.bitcast_convert_type(
                                w & cmask, jnp.float32)

                    pltpu.sync_copy(bufu, acc.at[didx.at[r]], add=True)
            return carry

        lax.fori_loop(0, nr, body, 0)
        plsc.subcore_barrier()

        def wout(k, carry):
            pltpu.sync_copy(acc.at[pl.ds(sid * NPS + k * CH, CH)], bufu)
            pltpu.sync_copy(bufu,
                            out_hbm.at[cid, pl.ds(sid * NPS + k * CH, CH)])
            return carry
        lax.fori_loop(0, NPS // CH, wout, 0)

    return _gather_sc, _scatter_sc


_BN = 1000
_BE = 640


def _project_body(nf, w1s, w1d, b1, ps, pd):
    x = nf[...]
    ps[...] = jnp.dot(x, w1s[...], preferred_element_type=jnp.float32) + b1[...]
    pd[...] = jnp.dot(x, w1d[...], preferred_element_type=jnp.float32)


def _unpack_pairs(p, nch):
    w = lax.bitcast_convert_type(p, jnp.uint32)
    lo = lax.bitcast_convert_type(
        lax.shift_left(w, jnp.uint32(16)), jnp.float32)
    hi = lax.bitcast_convert_type(w & jnp.uint32(0xFFFF0000), jnp.float32)
    return jnp.concatenate(
        [lo.reshape(nch, CH // 2, D), hi.reshape(nch, CH // 2, D)],
        axis=1).reshape(nch * CH, D)


def _pack_pairs(x, nch):
    x3 = x.reshape(nch, 2, CH // 2, D)
    def rtne(b):
        return b + jnp.uint32(0x7FFF) + (
            lax.shift_right_logical(b, jnp.uint32(16)) & jnp.uint32(1))
    lo = rtne(lax.bitcast_convert_type(x3[:, 0].reshape(nch * CH // 2, D),
                                       jnp.uint32))
    hi = rtne(lax.bitcast_convert_type(x3[:, 1].reshape(nch * CH // 2, D),
                                       jnp.uint32))
    packed = (lax.shift_right_logical(lo, jnp.uint32(16))
              | (hi & jnp.uint32(0xFFFF0000)))
    return lax.bitcast_convert_type(packed, jnp.float32)


def _edge_mlp(g, x, w1e, w2, b2, w3, b3):
    h = g + jnp.dot(x, w1e[...], preferred_element_type=jnp.float32)
    h = jnp.maximum(h, 0.0)
    h = jnp.dot(h, w2[...], preferred_element_type=jnp.float32) + b2[...]
    h = jnp.maximum(h, 0.0)
    return jnp.dot(h, w3[...], preferred_element_type=jnp.float32) + b3[...] + x


def _edge_body_first(gp, ef, w1e, w2, b2, w3, b3, out):
    nch = _BE // CH
    g = _unpack_pairs(gp[...], nch)
    out[...] = _pack_pairs(_edge_mlp(g, ef[...], w1e, w2, b2, w3, b3), nch)


def _edge_body(gp, efp, w1e, w2, b2, w3, b3, out):
    nch = _BE // CH
    g = _unpack_pairs(gp[...], nch)
    x = _unpack_pairs(efp[...], nch)
    out[...] = _pack_pairs(_edge_mlp(g, x, w1e, w2, b2, w3, b3), nch)


def _node_body(nf, p0, p1, w1a, w1b, b1, w2, b2, w3, b3,
               nw1s, nw1d, nb1, out, ps, pd):
    x = nf[...]
    agg = (p0[0] + p0[1]) + (p1[0] + p1[1])
    h = (jnp.dot(x, w1a[...], preferred_element_type=jnp.float32)
         + jnp.dot(agg, w1b[...], preferred_element_type=jnp.float32) + b1[...])
    h = jnp.maximum(h, 0.0)
    h = jnp.dot(h, w2[...], preferred_element_type=jnp.float32) + b2[...]
    h = jnp.maximum(h, 0.0)
    y = jnp.dot(h, w3[...], preferred_element_type=jnp.float32) + b3[...] + x
    out[...] = y
    ps[...] = jnp.dot(y, nw1s[...], preferred_element_type=jnp.float32) + nb1[...]
    pd[...] = jnp.dot(y, nw1d[...], preferred_element_type=jnp.float32)


def _row_spec(block):
    return pl.BlockSpec((block, D), lambda b: (b, 0))


def _half_spec(block):
    return pl.BlockSpec((block, D // 2), lambda b: (b, 0))


def _w_spec():
    return pl.BlockSpec((D, D), lambda b: (0, 0))


def _b_spec():
    return pl.BlockSpec((1, D), lambda b: (0, 0))


_project_tc = pl.pallas_call(
    _project_body,
    grid=(N // _BN,),
    in_specs=[_row_spec(_BN), _w_spec(), _w_spec(), _b_spec()],
    out_specs=[_row_spec(_BN), _row_spec(_BN)],
    out_shape=[jax.ShapeDtypeStruct((N, D), jnp.float32),
               jax.ShapeDtypeStruct((N, D), jnp.float32)],
)

def _make_edge_tc_first(off_blocks):
    return pl.pallas_call(
        _edge_body_first,
        grid=(ES // _BE,),
        in_specs=[_row_spec(_BE // 2),
                  pl.BlockSpec((_BE, D), lambda b: (b + off_blocks, 0)),
                  _w_spec(), _w_spec(), _b_spec(), _w_spec(), _b_spec()],
        out_specs=_row_spec(_BE // 2),
        out_shape=jax.ShapeDtypeStruct((ES // 2, D), jnp.float32),
    )


_edge_tc = pl.pallas_call(
    _edge_body,
    grid=(ES // _BE,),
    in_specs=[_row_spec(_BE // 2), _row_spec(_BE // 2),
              _w_spec(), _w_spec(), _b_spec(), _w_spec(), _b_spec()],
    out_specs=_row_spec(_BE // 2),
    out_shape=jax.ShapeDtypeStruct((ES // 2, D), jnp.float32),
)
_edge_tc_first = [_make_edge_tc_first(s * (ES // _BE)) for s in range(K_SL)]

_node_tc = pl.pallas_call(
    _node_body,
    grid=(N // _BN,),
    in_specs=[_row_spec(_BN),
              pl.BlockSpec((NC, _BN, D), lambda b: (0, b, 0)),
              pl.BlockSpec((NC, _BN, D), lambda b: (0, b, 0)),
              _w_spec(), _w_spec(), _b_spec(), _w_spec(), _b_spec(),
              _w_spec(), _b_spec(),
              _w_spec(), _w_spec(), _b_spec()],
    out_specs=[_row_spec(_BN), _row_spec(_BN), _row_spec(_BN)],
    out_shape=[jax.ShapeDtypeStruct((N, D), jnp.float32),
               jax.ShapeDtypeStruct((N, D), jnp.float32),
               jax.ShapeDtypeStruct((N, D), jnp.float32)],
)


def _worker_major(idx, s):
    sl = idx[s * ES:(s + 1) * ES]
    pad = NW * RPW * CH - ES
    sl = jnp.concatenate([sl, jnp.zeros((pad,), sl.dtype)])
    return sl.reshape(RPW, NW, CH).transpose(1, 0, 2)


@jax.jit
def kernel(node_features, edge_features, edge_index,
           edge_W1, edge_b1, edge_W2, edge_b2, edge_W3, edge_b3,
           node_W1, node_b1, node_W2, node_b2, node_W3, node_b3):
    gather_sc, scatter_sc = _sc_kernels()
    src3 = [_worker_major(edge_index[0], s) for s in range(K_SL)]
    dst3 = [_worker_major(edge_index[1], s) for s in range(K_SL)]
    zeros = jnp.zeros((CH, D), jnp.float32)

    nf = node_features
    efs = None
    nrounds = edge_W1.shape[0]
    w1 = edge_W1[0]
    ps, pd = _project_tc(nf, w1[:D], w1[D:2 * D], edge_b1[0].reshape(1, D))
    for i in range(nrounds):
        w1 = edge_W1[i]
        ew = (w1[2 * D:], edge_W2[i], edge_b2[i].reshape(1, D),
              edge_W3[i], edge_b3[i].reshape(1, D))
        gs = [gather_sc(ps, pd, src3[s], dst3[s]) for s in range(K_SL)]
        if efs is None:
            efs = [_edge_tc_first[s](gs[s], edge_features, *ew)
                   for s in range(K_SL)]
        else:
            efs = [_edge_tc(gs[s], efs[s], *ew) for s in range(K_SL)]
        parts = [scatter_sc(efs[s], dst3[s], zeros) for s in range(K_SL)]
        nw1 = node_W1[i]
        j = (i + 1) % nrounds
        w1n = edge_W1[j]
        nf, ps, pd = _node_tc(nf, parts[0], parts[1],
                              nw1[:D], nw1[D:], node_b1[i].reshape(1, D),
                              node_W2[i], node_b2[i].reshape(1, D),
                              node_W3[i], node_b3[i].reshape(1, D),
                              w1n[:D], w1n[D:2 * D],
                              edge_b1[j].reshape(1, D))
    return nf


# speedup vs baseline: 5.9676x; 1.4413x over previous
import functools

import jax
import jax.numpy as jnp
from jax import lax
from jax.experimental import pallas as pl
from jax.experimental.pallas import tpu as pltpu
from jax.experimental.pallas import tpu_sc as plsc

N = 10000
E = 160000
D = 128
NC = 2
NS = 16
NW = NC * NS
CH = 128
K_SL = 2
ES = E // K_SL
RS = ES // CH
RPW = -(-RS // NW)
NA = 10240
NPS = NA // NS


def _worker_nrows(wid):
    return jnp.where(wid < RS - (RPW - 1) * NW, RPW, RPW - 1)


@functools.cache
def _sc_kernels():
    mesh = plsc.VectorSubcoreMesh(core_axis_name="c", subcore_axis_name="s",
                                  num_cores=NC, num_subcores=NS)

    @functools.partial(
        pl.kernel,
        out_type=jax.ShapeDtypeStruct((ES // 2, D), jnp.float32),
        mesh=mesh,
        scratch_types=[
            pltpu.VMEM((RPW, CH), jnp.int32),
            pltpu.VMEM((RPW, CH), jnp.int32),
            pltpu.VMEM((2, CH, D), jnp.float32),
            pltpu.VMEM((2, CH, D), jnp.float32),
            pltpu.SemaphoreType.DMA,
            pltpu.SemaphoreType.DMA,
            pltpu.SemaphoreType.DMA,
            pltpu.SemaphoreType.DMA,
            pltpu.SemaphoreType.DMA,
            pltpu.SemaphoreType.DMA,
        ],
    )
    def _gather_sc(ps_hbm, pd_hbm, src_hbm, dst_hbm, g_hbm,
                   sidx, didx, bufa, bufb, sa0, sa1, sb0, sb1, sw0, sw1):
        wid = lax.axis_index("s") * NC + lax.axis_index("c")
        nr = _worker_nrows(wid)
        pltpu.sync_copy(src_hbm.at[wid], sidx)
        pltpu.sync_copy(dst_hbm.at[wid], didx)
        sas = [sa0, sa1]
        sbs = [sb0, sb1]
        sws = [sw0, sw1]
        HC = CH // 2

        def gath(r, s):
            pltpu.async_copy(ps_hbm.at[sidx.at[r]], bufa.at[s], sas[s])
            pltpu.async_copy(pd_hbm.at[didx.at[r]], bufb.at[s], sbs[s])

        def wait_gath(r, s):
            pltpu.make_async_copy(ps_hbm.at[sidx.at[r]], bufa.at[s],
                                  sas[s]).wait()
            pltpu.make_async_copy(pd_hbm.at[didx.at[r]], bufb.at[s],
                                  sbs[s]).wait()

        def out_ref(r, s):
            row = r * NW + wid
            return g_hbm.at[pl.ds(row * HC, HC)]

        def wait_w(r, s):
            pltpu.make_async_copy(bufb.at[s, pl.ds(0, HC)], out_ref(r, s),
                                  sws[s]).wait()

        c7fff = jnp.full((16,), 0x7FFF, jnp.int32)
        c16 = jnp.full((16,), 16, jnp.int32)
        c1 = jnp.full((16,), 1, jnp.int32)
        cmask = jnp.full((16,), -65536, jnp.int32)

        def rtne16(bits):
            return bits + c7fff + (lax.shift_right_logical(bits, c16) & c1)

        gath(0, 0)

        def step(r, carry):
            s = (r % 2).astype(jnp.int32)

            @pl.when(r + 1 < nr)
            def _():
                @pl.when(r >= 1)
                def _():
                    for s2 in (0, 1):
                        @pl.when(s2 != s)
                        def _():
                            wait_w(r - 1, s2)
                for s2 in (0, 1):
                    @pl.when(s2 != s)
                    def _():
                        gath(r + 1, s2)

            for s2 in (0, 1):
                @pl.when(s2 == s)
                def _():
                    wait_gath(r, s2)

                    @plsc.parallel_loop(0, HC, 1, unroll=2)
                    def _(i):
                        for j in range(D // 16):
                            c = pl.ds(j * 16, 16)
                            lo = (bufa[s2, i, c] + bufb[s2, i, c])
                            hi = (bufa[s2, i + HC, c] + bufb[s2, i + HC, c])
                            lo_u = rtne16(lax.bitcast_convert_type(lo, jnp.int32))
                            hi_u = rtne16(lax.bitcast_convert_type(hi, jnp.int32))
                            packed = (
                                lax.shift_right_logical(lo_u, c16)
                                | (hi_u & cmask))
                            bufb[s2, i, c] = lax.bitcast_convert_type(
                                packed, jnp.float32)

                    pltpu.async_copy(bufb.at[s2, pl.ds(0, HC)],
                                     out_ref(r, s2), sws[s2])
            return carry

        lax.fori_loop(0, nr, step, 0)

        def drain(r, carry):
            s = (r % 2).astype(jnp.int32)
            for s2 in (0, 1):
                @pl.when(s2 == s)
                def _():
                    wait_w(r, s2)
            return carry

        lax.fori_loop(jnp.maximum(nr - 2, 0), nr, drain, 0)

    @functools.partial(
        pl.kernel,
        out_type=jax.ShapeDtypeStruct((NC, NA, D), jnp.float32),
        mesh=mesh,
        scratch_types=[
            pltpu.VMEM((RPW, CH), jnp.int32),
            pltpu.VMEM((2, CH // 2, D), jnp.float32),
            pltpu.VMEM((CH, D), jnp.float32),
            pltpu.VMEM_SHARED((NA, D), jnp.float32),
            pltpu.SemaphoreType.DMA,
            pltpu.SemaphoreType.DMA,
        ],
    )
    def _scatter_sc(efp_hbm, dst_hbm, zeros_hbm, out_hbm, didx, bufp, bufu,
                    acc, sr0, sr1):
        cid = lax.axis_index("c")
        sid = lax.axis_index("s")
        wid = sid * NC + cid
        nr = _worker_nrows(wid)
        srs = [sr0, sr1]
        HC = CH // 2
        c16 = jnp.full((16,), 16, jnp.int32)
        cmask = jnp.full((16,), -65536, jnp.int32)

        pltpu.sync_copy(zeros_hbm, bufu)
        def zinit(k, carry):
            pltpu.sync_copy(bufu, acc.at[pl.ds(sid * NPS + k * CH, CH)])
            return carry
        lax.fori_loop(0, NPS // CH, zinit, 0)
        pltpu.sync_copy(dst_hbm.at[wid], didx)
        plsc.subcore_barrier()

        def rd(r, s):
            row = r * NW + wid
            pltpu.async_copy(efp_hbm.at[pl.ds(row * HC, HC)], bufp.at[s],
                             srs[s])

        def wait_rd(r, s):
            row = r * NW + wid
            pltpu.make_async_copy(efp_hbm.at[pl.ds(row * HC, HC)],
                                  bufp.at[s], srs[s]).wait()

        rd(0, 0)

        def body(r, carry):
            s = (r % 2).astype(jnp.int32)
            for s2 in (0, 1):
                @pl.when((s2 == s) & (r + 1 < nr))
                def _():
                    rd(r + 1, 1 - s2)
                @pl.when(s2 == s)
                def _():
                    wait_rd(r, s2)

                    @plsc.parallel_loop(0, HC, 1, unroll=2)
                    def _(i):
                        for j in range(D // 16):
                            c = pl.ds(j * 16, 16)
                            w = lax.bitcast_convert_type(bufp[s2, i, c],
                                                         jnp.int32)
                            bufu[i, c] = lax.bitcast_convert_type(
                                lax.shift_left(w, c16), jnp.float32)
                            bufu[i + HC, c] = lax.bitcast_convert_type(
                                w & cmask, jnp.float32)

                    pltpu.sync_copy(bufu, acc.at[didx.at[r]], add=True)
            return carry

        lax.fori_loop(0, nr, body, 0)
        plsc.subcore_barrier()

        def wout(k, carry):
            pltpu.sync_copy(acc.at[pl.ds(sid * NPS + k * CH, CH)], bufu)
            pltpu.sync_copy(bufu,
                            out_hbm.at[cid, pl.ds(sid * NPS + k * CH, CH)])
            return carry
        lax.fori_loop(0, NPS // CH, wout, 0)

    return _gather_sc, _scatter_sc


_BN = 1000
_BE = 3200


def _project_body(nf, w1s, w1d, b1, ps, pd):
    x = nf[...]
    ps[...] = jnp.dot(x, w1s[...], preferred_element_type=jnp.float32) + b1[...]
    pd[...] = jnp.dot(x, w1d[...], preferred_element_type=jnp.float32)


def _unpack_pairs(p, nch):
    w = lax.bitcast_convert_type(p, jnp.uint32)
    lo = lax.bitcast_convert_type(
        lax.shift_left(w, jnp.uint32(16)), jnp.float32)
    hi = lax.bitcast_convert_type(w & jnp.uint32(0xFFFF0000), jnp.float32)
    return jnp.concatenate(
        [lo.reshape(nch, CH // 2, D), hi.reshape(nch, CH // 2, D)],
        axis=1).reshape(nch * CH, D)


def _pack_pairs(x, nch):
    x3 = x.reshape(nch, 2, CH // 2, D)
    def rtne(b):
        return b + jnp.uint32(0x7FFF) + (
            lax.shift_right_logical(b, jnp.uint32(16)) & jnp.uint32(1))
    lo = rtne(lax.bitcast_convert_type(x3[:, 0].reshape(nch * CH // 2, D),
                                       jnp.uint32))
    hi = rtne(lax.bitcast_convert_type(x3[:, 1].reshape(nch * CH // 2, D),
                                       jnp.uint32))
    packed = (lax.shift_right_logical(lo, jnp.uint32(16))
              | (hi & jnp.uint32(0xFFFF0000)))
    return lax.bitcast_convert_type(packed, jnp.float32)


def _edge_mlp(g, x, w1e, w2, b2, w3, b3):
    h = g + jnp.dot(x, w1e[...], preferred_element_type=jnp.float32)
    h = jnp.maximum(h, 0.0)
    h = jnp.dot(h, w2[...], preferred_element_type=jnp.float32) + b2[...]
    h = jnp.maximum(h, 0.0)
    return jnp.dot(h, w3[...], preferred_element_type=jnp.float32) + b3[...] + x


def _edge_body_first(gp, ef, w1e, w2, b2, w3, b3, out):
    nch = _BE // CH
    g = _unpack_pairs(gp[...], nch)
    out[...] = _pack_pairs(_edge_mlp(g, ef[...], w1e, w2, b2, w3, b3), nch)


def _edge_body(gp, efp, w1e, w2, b2, w3, b3, out):
    nch = _BE // CH
    g = _unpack_pairs(gp[...], nch)
    x = _unpack_pairs(efp[...], nch)
    out[...] = _pack_pairs(_edge_mlp(g, x, w1e, w2, b2, w3, b3), nch)


def _node_body(nf, p0, p1, w1a, w1b, b1, w2, b2, w3, b3,
               nw1s, nw1d, nb1, out, ps, pd):
    x = nf[...]
    agg = (p0[0] + p0[1]) + (p1[0] + p1[1])
    h = (jnp.dot(x, w1a[...], preferred_element_type=jnp.float32)
         + jnp.dot(agg, w1b[...], preferred_element_type=jnp.float32) + b1[...])
    h = jnp.maximum(h, 0.0)
    h = jnp.dot(h, w2[...], preferred_element_type=jnp.float32) + b2[...]
    h = jnp.maximum(h, 0.0)
    y = jnp.dot(h, w3[...], preferred_element_type=jnp.float32) + b3[...] + x
    out[...] = y
    ps[...] = jnp.dot(y, nw1s[...], preferred_element_type=jnp.float32) + nb1[...]
    pd[...] = jnp.dot(y, nw1d[...], preferred_element_type=jnp.float32)


def _row_spec(block):
    return pl.BlockSpec((block, D), lambda b: (b, 0))


def _half_spec(block):
    return pl.BlockSpec((block, D // 2), lambda b: (b, 0))


def _w_spec():
    return pl.BlockSpec((D, D), lambda b: (0, 0))


def _b_spec():
    return pl.BlockSpec((1, D), lambda b: (0, 0))


_project_tc = pl.pallas_call(
    _project_body,
    grid=(N // _BN,),
    in_specs=[_row_spec(_BN), _w_spec(), _w_spec(), _b_spec()],
    out_specs=[_row_spec(_BN), _row_spec(_BN)],
    out_shape=[jax.ShapeDtypeStruct((N, D), jnp.float32),
               jax.ShapeDtypeStruct((N, D), jnp.float32)],
)

def _make_edge_tc_first(off_blocks):
    return pl.pallas_call(
        _edge_body_first,
        grid=(ES // _BE,),
        in_specs=[_row_spec(_BE // 2),
                  pl.BlockSpec((_BE, D), lambda b: (b + off_blocks, 0)),
                  _w_spec(), _w_spec(), _b_spec(), _w_spec(), _b_spec()],
        out_specs=_row_spec(_BE // 2),
        out_shape=jax.ShapeDtypeStruct((ES // 2, D), jnp.float32),
    )


_edge_tc = pl.pallas_call(
    _edge_body,
    grid=(ES // _BE,),
    in_specs=[_row_spec(_BE // 2), _row_spec(_BE // 2),
              _w_spec(), _w_spec(), _b_spec(), _w_spec(), _b_spec()],
    out_specs=_row_spec(_BE // 2),
    out_shape=jax.ShapeDtypeStruct((ES // 2, D), jnp.float32),
)
_edge_tc_first = [_make_edge_tc_first(s * (ES // _BE)) for s in range(K_SL)]

_node_tc = pl.pallas_call(
    _node_body,
    grid=(N // _BN,),
    in_specs=[_row_spec(_BN),
              pl.BlockSpec((NC, _BN, D), lambda b: (0, b, 0)),
              pl.BlockSpec((NC, _BN, D), lambda b: (0, b, 0)),
              _w_spec(), _w_spec(), _b_spec(), _w_spec(), _b_spec(),
              _w_spec(), _b_spec(),
              _w_spec(), _w_spec(), _b_spec()],
    out_specs=[_row_spec(_BN), _row_spec(_BN), _row_spec(_BN)],
    out_shape=[jax.ShapeDtypeStruct((N, D), jnp.float32),
               jax.ShapeDtypeStruct((N, D), jnp.float32),
               jax.ShapeDtypeStruct((N, D), jnp.float32)],
)


def _worker_major(idx, s):
    sl = idx[s * ES:(s + 1) * ES]
    pad = NW * RPW * CH - ES
    sl = jnp.concatenate([sl, jnp.zeros((pad,), sl.dtype)])
    return sl.reshape(RPW, NW, CH).transpose(1, 0, 2)


@jax.jit
def kernel(node_features, edge_features, edge_index,
           edge_W1, edge_b1, edge_W2, edge_b2, edge_W3, edge_b3,
           node_W1, node_b1, node_W2, node_b2, node_W3, node_b3):
    gather_sc, scatter_sc = _sc_kernels()
    src3 = [_worker_major(edge_index[0], s) for s in range(K_SL)]
    dst3 = [_worker_major(edge_index[1], s) for s in range(K_SL)]
    zeros = jnp.zeros((CH, D), jnp.float32)

    nf = node_features
    efs = None
    nrounds = edge_W1.shape[0]
    w1 = edge_W1[0]
    ps, pd = _project_tc(nf, w1[:D], w1[D:2 * D], edge_b1[0].reshape(1, D))
    for i in range(nrounds):
        w1 = edge_W1[i]
        ew = (w1[2 * D:], edge_W2[i], edge_b2[i].reshape(1, D),
              edge_W3[i], edge_b3[i].reshape(1, D))
        gs = [gather_sc(ps, pd, src3[s], dst3[s]) for s in range(K_SL)]
        if efs is None:
            efs = [_edge_tc_first[s](gs[s], edge_features, *ew)
                   for s in range(K_SL)]
        else:
            efs = [_edge_tc(gs[s], efs[s], *ew) for s in range(K_SL)]
        parts = [scatter_sc(efs[s], dst3[s], zeros) for s in range(K_SL)]
        nw1 = node_W1[i]
        j = (i + 1) % nrounds
        w1n = edge_W1[j]
        nf, ps, pd = _node_tc(nf, parts[0], parts[1],
                              nw1[:D], nw1[D:], node_b1[i].reshape(1, D),
                              node_W2[i], node_b2[i].reshape(1, D),
                              node_W3[i], node_b3[i].reshape(1, D),
                              w1n[:D], w1n[D:2 * D],
                              edge_b1[j].reshape(1, D))
    return nf


# speedup vs baseline: 6.0382x; 1.0118x over previous
import functools

import jax
import jax.numpy as jnp
from jax import lax
from jax.experimental import pallas as pl
from jax.experimental.pallas import tpu as pltpu
from jax.experimental.pallas import tpu_sc as plsc

N = 10000
E = 160000
D = 128
NC = 2
NS = 16
NW = NC * NS
CH = 128
K_SL = 2
ES = E // K_SL
RS = ES // CH
RPW = -(-RS // NW)
NA = 10240
NPS = NA // NS


def _worker_nrows(wid):
    return jnp.where(wid < RS - (RPW - 1) * NW, RPW, RPW - 1)


@functools.cache
def _sc_kernels():
    mesh = plsc.VectorSubcoreMesh(core_axis_name="c", subcore_axis_name="s",
                                  num_cores=NC, num_subcores=NS)

    @functools.partial(
        pl.kernel,
        out_type=jax.ShapeDtypeStruct((ES // 2, D), jnp.float32),
        mesh=mesh,
        scratch_types=[
            pltpu.VMEM((RPW, CH), jnp.int32),
            pltpu.VMEM((RPW, CH), jnp.int32),
            pltpu.VMEM((2, CH, D), jnp.float32),
            pltpu.VMEM((2, CH, D), jnp.float32),
            pltpu.SemaphoreType.DMA,
            pltpu.SemaphoreType.DMA,
            pltpu.SemaphoreType.DMA,
            pltpu.SemaphoreType.DMA,
            pltpu.SemaphoreType.DMA,
            pltpu.SemaphoreType.DMA,
        ],
    )
    def _gather_sc(ps_hbm, pd_hbm, src_hbm, dst_hbm, g_hbm,
                   sidx, didx, bufa, bufb, sa0, sa1, sb0, sb1, sw0, sw1):
        wid = lax.axis_index("s") * NC + lax.axis_index("c")
        nr = _worker_nrows(wid)
        pltpu.sync_copy(src_hbm.at[wid], sidx)
        pltpu.sync_copy(dst_hbm.at[wid], didx)
        sas = [sa0, sa1]
        sbs = [sb0, sb1]
        sws = [sw0, sw1]
        HC = CH // 2

        def gath(r, s):
            pltpu.async_copy(ps_hbm.at[sidx.at[r]], bufa.at[s], sas[s])
            pltpu.async_copy(pd_hbm.at[didx.at[r]], bufb.at[s], sbs[s])

        def wait_gath(r, s):
            pltpu.make_async_copy(ps_hbm.at[sidx.at[r]], bufa.at[s],
                                  sas[s]).wait()
            pltpu.make_async_copy(pd_hbm.at[didx.at[r]], bufb.at[s],
                                  sbs[s]).wait()

        def out_ref(r, s):
            row = r * NW + wid
            return g_hbm.at[pl.ds(row * HC, HC)]

        def wait_w(r, s):
            pltpu.make_async_copy(bufb.at[s, pl.ds(0, HC)], out_ref(r, s),
                                  sws[s]).wait()

        c7fff = jnp.full((16,), 0x7FFF, jnp.int32)
        c16 = jnp.full((16,), 16, jnp.int32)
        c1 = jnp.full((16,), 1, jnp.int32)
        cmask = jnp.full((16,), -65536, jnp.int32)

        def rtne16(bits):
            return bits + c7fff + (lax.shift_right_logical(bits, c16) & c1)

        gath(0, 0)

        def step(r, carry):
            s = (r % 2).astype(jnp.int32)

            @pl.when(r + 1 < nr)
            def _():
                @pl.when(r >= 1)
                def _():
                    for s2 in (0, 1):
                        @pl.when(s2 != s)
                        def _():
                            wait_w(r - 1, s2)
                for s2 in (0, 1):
                    @pl.when(s2 != s)
                    def _():
                        gath(r + 1, s2)

            for s2 in (0, 1):
                @pl.when(s2 == s)
                def _():
                    wait_gath(r, s2)

                    @plsc.parallel_loop(0, HC, 1, unroll=2)
                    def _(i):
                        for j in range(D // 16):
                            c = pl.ds(j * 16, 16)
                            lo = (bufa[s2, i, c] + bufb[s2, i, c])
                            hi = (bufa[s2, i + HC, c] + bufb[s2, i + HC, c])
                            lo_u = rtne16(lax.bitcast_convert_type(lo, jnp.int32))
                            hi_u = rtne16(lax.bitcast_convert_type(hi, jnp.int32))
                            packed = (
                                lax.shift_right_logical(lo_u, c16)
                                | (hi_u & cmask))
                            bufb[s2, i, c] = lax.bitcast_convert_type(
                                packed, jnp.float32)

                    pltpu.async_copy(bufb.at[s2, pl.ds(0, HC)],
                                     out_ref(r, s2), sws[s2])
            return carry

        lax.fori_loop(0, nr, step, 0)

        def drain(r, carry):
            s = (r % 2).astype(jnp.int32)
            for s2 in (0, 1):
                @pl.when(s2 == s)
                def _():
                    wait_w(r, s2)
            return carry

        lax.fori_loop(jnp.maximum(nr - 2, 0), nr, drain, 0)

    @functools.partial(
        pl.kernel,
        out_type=jax.ShapeDtypeStruct((NC, NA, D), jnp.float32),
        mesh=mesh,
        scratch_types=[
            pltpu.VMEM((RPW, CH), jnp.int32),
            pltpu.VMEM((2, CH // 2, D), jnp.float32),
            pltpu.VMEM((CH, D), jnp.float32),
            pltpu.VMEM_SHARED((NA, D), jnp.float32),
            pltpu.SemaphoreType.DMA,
            pltpu.SemaphoreType.DMA,
        ],
    )
    def _scatter_sc(efp_hbm, dst_hbm, zeros_hbm, out_hbm, didx, bufp, bufu,
                    acc, sr0, sr1):
        cid = lax.axis_index("c")
        sid = lax.axis_index("s")
        wid = sid * NC + cid
        nr = _worker_nrows(wid)
        srs = [sr0, sr1]
        HC = CH // 2
        c16 = jnp.full((16,), 16, jnp.int32)
        cmask = jnp.full((16,), -65536, jnp.int32)

        pltpu.sync_copy(zeros_hbm, bufu)
        def zinit(k, carry):
            pltpu.sync_copy(bufu, acc.at[pl.ds(sid * NPS + k * CH, CH)])
            return carry
        lax.fori_loop(0, NPS // CH, zinit, 0)
        pltpu.sync_copy(dst_hbm.at[wid], didx)
        plsc.subcore_barrier()

        def rd(r, s):
            row = r * NW + wid
            pltpu.async_copy(efp_hbm.at[pl.ds(row * HC, HC)], bufp.at[s],
                             srs[s])

        def wait_rd(r, s):
            row = r * NW + wid
            pltpu.make_async_copy(efp_hbm.at[pl.ds(row * HC, HC)],
                                  bufp.at[s], srs[s]).wait()

        rd(0, 0)

        def body(r, carry):
            s = (r % 2).astype(jnp.int32)
            for s2 in (0, 1):
                @pl.when((s2 == s) & (r + 1 < nr))
                def _():
                    rd(r + 1, 1 - s2)
                @pl.when(s2 == s)
                def _():
                    wait_rd(r, s2)

                    @plsc.parallel_loop(0, HC, 1, unroll=2)
                    def _(i):
                        for j in range(D // 16):
                            c = pl.ds(j * 16, 16)
                            w = lax.bitcast_convert_type(bufp[s2, i, c],
                                                         jnp.int32)
                            bufu[i, c] = lax.bitcast_convert_type(
                                lax.shift_left(w, c16), jnp.float32)
                            bufu[i + HC, c] = lax.bitcast_convert_type(
                                w & cmask, jnp.float32)

                    pltpu.sync_copy(bufu, acc.at[didx.at[r]], add=True)
            return carry

        lax.fori_loop(0, nr, body, 0)
        plsc.subcore_barrier()

        def wout(k, carry):
            pltpu.sync_copy(acc.at[pl.ds(sid * NPS + k * CH, CH)], bufu)
            pltpu.sync_copy(bufu,
                            out_hbm.at[cid, pl.ds(sid * NPS + k * CH, CH)])
            return carry
        lax.fori_loop(0, NPS // CH, wout, 0)

    return _gather_sc, _scatter_sc


_BN = 2000
_BE = 16000


def _project_body(nf, w1s, w1d, b1, ps, pd):
    x = nf[...]
    ps[...] = jnp.dot(x, w1s[...], preferred_element_type=jnp.float32) + b1[...]
    pd[...] = jnp.dot(x, w1d[...], preferred_element_type=jnp.float32)


def _unpack_pairs(p, nch):
    w = lax.bitcast_convert_type(p, jnp.uint32)
    lo = lax.bitcast_convert_type(
        lax.shift_left(w, jnp.uint32(16)), jnp.float32)
    hi = lax.bitcast_convert_type(w & jnp.uint32(0xFFFF0000), jnp.float32)
    return jnp.concatenate(
        [lo.reshape(nch, CH // 2, D), hi.reshape(nch, CH // 2, D)],
        axis=1).reshape(nch * CH, D)


def _pack_pairs(x, nch):
    x3 = x.reshape(nch, 2, CH // 2, D)
    def rtne(b):
        return b + jnp.uint32(0x7FFF) + (
            lax.shift_right_logical(b, jnp.uint32(16)) & jnp.uint32(1))
    lo = rtne(lax.bitcast_convert_type(x3[:, 0].reshape(nch * CH // 2, D),
                                       jnp.uint32))
    hi = rtne(lax.bitcast_convert_type(x3[:, 1].reshape(nch * CH // 2, D),
                                       jnp.uint32))
    packed = (lax.shift_right_logical(lo, jnp.uint32(16))
              | (hi & jnp.uint32(0xFFFF0000)))
    return lax.bitcast_convert_type(packed, jnp.float32)


def _edge_mlp(g, x, w1e, w2, b2, w3, b3):
    h = g + jnp.dot(x, w1e[...], preferred_element_type=jnp.float32)
    h = jnp.maximum(h, 0.0)
    h = jnp.dot(h, w2[...], preferred_element_type=jnp.float32) + b2[...]
    h = jnp.maximum(h, 0.0)
    return jnp.dot(h, w3[...], preferred_element_type=jnp.float32) + b3[...] + x


def _edge_body_first(gp, ef, w1e, w2, b2, w3, b3, out):
    nch = _BE // CH
    g = _unpack_pairs(gp[...], nch)
    out[...] = _pack_pairs(_edge_mlp(g, ef[...], w1e, w2, b2, w3, b3), nch)


def _edge_body(gp, efp, w1e, w2, b2, w3, b3, out):
    nch = _BE // CH
    g = _unpack_pairs(gp[...], nch)
    x = _unpack_pairs(efp[...], nch)
    out[...] = _pack_pairs(_edge_mlp(g, x, w1e, w2, b2, w3, b3), nch)


def _node_body(nf, p0, p1, w1a, w1b, b1, w2, b2, w3, b3,
               nw1s, nw1d, nb1, out, ps, pd):
    x = nf[...]
    agg = (p0[0] + p0[1]) + (p1[0] + p1[1])
    h = (jnp.dot(x, w1a[...], preferred_element_type=jnp.float32)
         + jnp.dot(agg, w1b[...], preferred_element_type=jnp.float32) + b1[...])
    h = jnp.maximum(h, 0.0)
    h = jnp.dot(h, w2[...], preferred_element_type=jnp.float32) + b2[...]
    h = jnp.maximum(h, 0.0)
    y = jnp.dot(h, w3[...], preferred_element_type=jnp.float32) + b3[...] + x
    out[...] = y
    ps[...] = jnp.dot(y, nw1s[...], preferred_element_type=jnp.float32) + nb1[...]
    pd[...] = jnp.dot(y, nw1d[...], preferred_element_type=jnp.float32)


def _row_spec(block):
    return pl.BlockSpec((block, D), lambda b: (b, 0))


def _half_spec(block):
    return pl.BlockSpec((block, D // 2), lambda b: (b, 0))


def _w_spec():
    return pl.BlockSpec((D, D), lambda b: (0, 0))


def _b_spec():
    return pl.BlockSpec((1, D), lambda b: (0, 0))


_project_tc = pl.pallas_call(
    _project_body,
    grid=(N // _BN,),
    in_specs=[_row_spec(_BN), _w_spec(), _w_spec(), _b_spec()],
    out_specs=[_row_spec(_BN), _row_spec(_BN)],
    out_shape=[jax.ShapeDtypeStruct((N, D), jnp.float32),
               jax.ShapeDtypeStruct((N, D), jnp.float32)],
)

def _make_edge_tc_first(off_blocks):
    return pl.pallas_call(
        _edge_body_first,
        grid=(ES // _BE,),
        in_specs=[_row_spec(_BE // 2),
                  pl.BlockSpec((_BE, D), lambda b: (b + off_blocks, 0)),
                  _w_spec(), _w_spec(), _b_spec(), _w_spec(), _b_spec()],
        out_specs=_row_spec(_BE // 2),
        out_shape=jax.ShapeDtypeStruct((ES // 2, D), jnp.float32),
    )


_edge_tc = pl.pallas_call(
    _edge_body,
    grid=(ES // _BE,),
    in_specs=[_row_spec(_BE // 2), _row_spec(_BE // 2),
              _w_spec(), _w_spec(), _b_spec(), _w_spec(), _b_spec()],
    out_specs=_row_spec(_BE // 2),
    out_shape=jax.ShapeDtypeStruct((ES // 2, D), jnp.float32),
)
_edge_tc_first = [_make_edge_tc_first(s * (ES // _BE)) for s in range(K_SL)]

_node_tc = pl.pallas_call(
    _node_body,
    grid=(N // _BN,),
    in_specs=[_row_spec(_BN),
              pl.BlockSpec((NC, _BN, D), lambda b: (0, b, 0)),
              pl.BlockSpec((NC, _BN, D), lambda b: (0, b, 0)),
              _w_spec(), _w_spec(), _b_spec(), _w_spec(), _b_spec(),
              _w_spec(), _b_spec(),
              _w_spec(), _w_spec(), _b_spec()],
    out_specs=[_row_spec(_BN), _row_spec(_BN), _row_spec(_BN)],
    out_shape=[jax.ShapeDtypeStruct((N, D), jnp.float32),
               jax.ShapeDtypeStruct((N, D), jnp.float32),
               jax.ShapeDtypeStruct((N, D), jnp.float32)],
)


def _worker_major(idx, s):
    sl = idx[s * ES:(s + 1) * ES]
    pad = NW * RPW * CH - ES
    sl = jnp.concatenate([sl, jnp.zeros((pad,), sl.dtype)])
    return sl.reshape(RPW, NW, CH).transpose(1, 0, 2)


@jax.jit
def kernel(node_features, edge_features, edge_index,
           edge_W1, edge_b1, edge_W2, edge_b2, edge_W3, edge_b3,
           node_W1, node_b1, node_W2, node_b2, node_W3, node_b3):
    gather_sc, scatter_sc = _sc_kernels()
    src3 = [_worker_major(edge_index[0], s) for s in range(K_SL)]
    dst3 = [_worker_major(edge_index[1], s) for s in range(K_SL)]
    zeros = jnp.zeros((CH, D), jnp.float32)

    nf = node_features
    efs = None
    nrounds = edge_W1.shape[0]
    w1 = edge_W1[0]
    ps, pd = _project_tc(nf, w1[:D], w1[D:2 * D], edge_b1[0].reshape(1, D))
    for i in range(nrounds):
        w1 = edge_W1[i]
        ew = (w1[2 * D:], edge_W2[i], edge_b2[i].reshape(1, D),
              edge_W3[i], edge_b3[i].reshape(1, D))
        gs = [gather_sc(ps, pd, src3[s], dst3[s]) for s in range(K_SL)]
        if efs is None:
            efs = [_edge_tc_first[s](gs[s], edge_features, *ew)
                   for s in range(K_SL)]
        else:
            efs = [_edge_tc(gs[s], efs[s], *ew) for s in range(K_SL)]
        parts = [scatter_sc(efs[s], dst3[s], zeros) for s in range(K_SL)]
        nw1 = node_W1[i]
        j = (i + 1) % nrounds
        w1n = edge_W1[j]
        nf, ps, pd = _node_tc(nf, parts[0], parts[1],
                              nw1[:D], nw1[D:], node_b1[i].reshape(1, D),
                              node_W2[i], node_b2[i].reshape(1, D),
                              node_W3[i], node_b3[i].reshape(1, D),
                              w1n[:D], w1n[D:2 * D],
                              edge_b1[j].reshape(1, D))
    return nf


# speedup vs baseline: 6.0396x; 1.0002x over previous
import functools

import jax
import jax.numpy as jnp
from jax import lax
from jax.experimental import pallas as pl
from jax.experimental.pallas import tpu as pltpu
from jax.experimental.pallas import tpu_sc as plsc

N = 10000
E = 160000
D = 128
NC = 2
NS = 16
NW = NC * NS
CH = 128
K_SL = 2
ES = E // K_SL
RS = ES // CH
RPW = -(-RS // NW)
NA = 10240
NPS = NA // NS


def _worker_nrows(wid):
    return jnp.where(wid < RS - (RPW - 1) * NW, RPW, RPW - 1)


@functools.cache
def _sc_kernels():
    mesh = plsc.VectorSubcoreMesh(core_axis_name="c", subcore_axis_name="s",
                                  num_cores=NC, num_subcores=NS)

    @functools.partial(
        pl.kernel,
        out_type=jax.ShapeDtypeStruct((ES // 2, D), jnp.float32),
        mesh=mesh,
        scratch_types=[
            pltpu.VMEM((RPW, CH), jnp.int32),
            pltpu.VMEM((RPW, CH), jnp.int32),
            pltpu.VMEM((2, CH, D), jnp.float32),
            pltpu.VMEM((2, CH, D), jnp.float32),
            pltpu.SemaphoreType.DMA,
            pltpu.SemaphoreType.DMA,
            pltpu.SemaphoreType.DMA,
            pltpu.SemaphoreType.DMA,
            pltpu.SemaphoreType.DMA,
            pltpu.SemaphoreType.DMA,
        ],
    )
    def _gather_sc(ps_hbm, pd_hbm, src_hbm, dst_hbm, g_hbm,
                   sidx, didx, bufa, bufb, sa0, sa1, sb0, sb1, sw0, sw1):
        wid = lax.axis_index("s") * NC + lax.axis_index("c")
        nr = _worker_nrows(wid)
        pltpu.sync_copy(src_hbm.at[wid], sidx)
        pltpu.sync_copy(dst_hbm.at[wid], didx)
        sas = [sa0, sa1]
        sbs = [sb0, sb1]
        sws = [sw0, sw1]
        HC = CH // 2

        def gath(r, s):
            pltpu.async_copy(ps_hbm.at[sidx.at[r]], bufa.at[s], sas[s])
            pltpu.async_copy(pd_hbm.at[didx.at[r]], bufb.at[s], sbs[s])

        def wait_gath(r, s):
            pltpu.make_async_copy(ps_hbm.at[sidx.at[r]], bufa.at[s],
                                  sas[s]).wait()
            pltpu.make_async_copy(pd_hbm.at[didx.at[r]], bufb.at[s],
                                  sbs[s]).wait()

        def out_ref(r, s):
            row = r * NW + wid
            return g_hbm.at[pl.ds(row * HC, HC)]

        def wait_w(r, s):
            pltpu.make_async_copy(bufb.at[s, pl.ds(0, HC)], out_ref(r, s),
                                  sws[s]).wait()

        c7fff = jnp.full((16,), 0x7FFF, jnp.int32)
        c16 = jnp.full((16,), 16, jnp.int32)
        c1 = jnp.full((16,), 1, jnp.int32)
        cmask = jnp.full((16,), -65536, jnp.int32)

        def rtne16(bits):
            return bits + c7fff + (lax.shift_right_logical(bits, c16) & c1)

        gath(0, 0)

        def step(r, carry):
            s = (r % 2).astype(jnp.int32)

            @pl.when(r + 1 < nr)
            def _():
                @pl.when(r >= 1)
                def _():
                    for s2 in (0, 1):
                        @pl.when(s2 != s)
                        def _():
                            wait_w(r - 1, s2)
                for s2 in (0, 1):
                    @pl.when(s2 != s)
                    def _():
                        gath(r + 1, s2)

            for s2 in (0, 1):
                @pl.when(s2 == s)
                def _():
                    wait_gath(r, s2)

                    @plsc.parallel_loop(0, HC, 1, unroll=2)
                    def _(i):
                        for j in range(D // 16):
                            c = pl.ds(j * 16, 16)
                            lo = (bufa[s2, i, c] + bufb[s2, i, c])
                            hi = (bufa[s2, i + HC, c] + bufb[s2, i + HC, c])
                            lo_u = rtne16(lax.bitcast_convert_type(lo, jnp.int32))
                            hi_u = rtne16(lax.bitcast_convert_type(hi, jnp.int32))
                            packed = (
                                lax.shift_right_logical(lo_u, c16)
                                | (hi_u & cmask))
                            bufb[s2, i, c] = lax.bitcast_convert_type(
                                packed, jnp.float32)

                    pltpu.async_copy(bufb.at[s2, pl.ds(0, HC)],
                                     out_ref(r, s2), sws[s2])
            return carry

        lax.fori_loop(0, nr, step, 0)

        def drain(r, carry):
            s = (r % 2).astype(jnp.int32)
            for s2 in (0, 1):
                @pl.when(s2 == s)
                def _():
                    wait_w(r, s2)
            return carry

        lax.fori_loop(jnp.maximum(nr - 2, 0), nr, drain, 0)

    @functools.partial(
        pl.kernel,
        out_type=jax.ShapeDtypeStruct((NC, NA, D), jnp.float32),
        mesh=mesh,
        scratch_types=[
            pltpu.VMEM((RPW, CH), jnp.int32),
            pltpu.VMEM((2, CH // 2, D), jnp.float32),
            pltpu.VMEM((CH, D), jnp.float32),
            pltpu.VMEM_SHARED((NA, D), jnp.float32),
            pltpu.SemaphoreType.DMA,
            pltpu.SemaphoreType.DMA,
        ],
    )
    def _scatter_sc(efp_hbm, dst_hbm, zeros_hbm, out_hbm, didx, bufp, bufu,
                    acc, sr0, sr1):
        cid = lax.axis_index("c")
        sid = lax.axis_index("s")
        wid = sid * NC + cid
        nr = _worker_nrows(wid)
        srs = [sr0, sr1]
        HC = CH // 2
        c16 = jnp.full((16,), 16, jnp.int32)
        cmask = jnp.full((16,), -65536, jnp.int32)

        pltpu.sync_copy(zeros_hbm, bufu)
        def zinit(k, carry):
            pltpu.sync_copy(bufu, acc.at[pl.ds(sid * NPS + k * CH, CH)])
            return carry
        lax.fori_loop(0, NPS // CH, zinit, 0)
        pltpu.sync_copy(dst_hbm.at[wid], didx)
        plsc.subcore_barrier()

        def rd(r, s):
            row = r * NW + wid
            pltpu.async_copy(efp_hbm.at[pl.ds(row * HC, HC)], bufp.at[s],
                             srs[s])

        def wait_rd(r, s):
            row = r * NW + wid
            pltpu.make_async_copy(efp_hbm.at[pl.ds(row * HC, HC)],
                                  bufp.at[s], srs[s]).wait()

        rd(0, 0)

        def body(r, carry):
            s = (r % 2).astype(jnp.int32)
            for s2 in (0, 1):
                @pl.when((s2 == s) & (r + 1 < nr))
                def _():
                    rd(r + 1, 1 - s2)
                @pl.when(s2 == s)
                def _():
                    wait_rd(r, s2)

                    @plsc.parallel_loop(0, HC, 1, unroll=2)
                    def _(i):
                        for j in range(D // 16):
                            c = pl.ds(j * 16, 16)
                            w = lax.bitcast_convert_type(bufp[s2, i, c],
                                                         jnp.int32)
                            bufu[i, c] = lax.bitcast_convert_type(
                                lax.shift_left(w, c16), jnp.float32)
                            bufu[i + HC, c] = lax.bitcast_convert_type(
                                w & cmask, jnp.float32)

                    pltpu.sync_copy(bufu, acc.at[didx.at[r]], add=True)
            return carry

        lax.fori_loop(0, nr, body, 0)
        plsc.subcore_barrier()

        def wout(k, carry):
            pltpu.sync_copy(acc.at[pl.ds(sid * NPS + k * CH, CH)], bufu)
            pltpu.sync_copy(bufu,
                            out_hbm.at[cid, pl.ds(sid * NPS + k * CH, CH)])
            return carry
        lax.fori_loop(0, NPS // CH, wout, 0)

    return _gather_sc, _scatter_sc


_BN = 2000
_BE = 3200


def _project_body(nf, w1s, w1d, b1, ps, pd):
    x = nf[...]
    ps[...] = jnp.dot(x, w1s[...], preferred_element_type=jnp.float32) + b1[...]
    pd[...] = jnp.dot(x, w1d[...], preferred_element_type=jnp.float32)


def _unpack_pairs(p, nch):
    w = lax.bitcast_convert_type(p, jnp.uint32)
    lo = lax.bitcast_convert_type(
        lax.shift_left(w, jnp.uint32(16)), jnp.float32)
    hi = lax.bitcast_convert_type(w & jnp.uint32(0xFFFF0000), jnp.float32)
    return jnp.concatenate(
        [lo.reshape(nch, CH // 2, D), hi.reshape(nch, CH // 2, D)],
        axis=1).reshape(nch * CH, D)


def _pack_pairs(x, nch):
    x3 = x.reshape(nch, 2, CH // 2, D)
    def rtne(b):
        return b + jnp.uint32(0x7FFF) + (
            lax.shift_right_logical(b, jnp.uint32(16)) & jnp.uint32(1))
    lo = rtne(lax.bitcast_convert_type(x3[:, 0].reshape(nch * CH // 2, D),
                                       jnp.uint32))
    hi = rtne(lax.bitcast_convert_type(x3[:, 1].reshape(nch * CH // 2, D),
                                       jnp.uint32))
    packed = (lax.shift_right_logical(lo, jnp.uint32(16))
              | (hi & jnp.uint32(0xFFFF0000)))
    return lax.bitcast_convert_type(packed, jnp.float32)


def _edge_mlp(g, x, w1e, w2, b2, w3, b3):
    h = g + jnp.dot(x, w1e[...], preferred_element_type=jnp.float32)
    h = jnp.maximum(h, 0.0)
    h = jnp.dot(h, w2[...], preferred_element_type=jnp.float32) + b2[...]
    h = jnp.maximum(h, 0.0)
    return jnp.dot(h, w3[...], preferred_element_type=jnp.float32) + b3[...] + x


def _edge_body_first(gp, ef, w1e, w2, b2, w3, b3, out):
    nch = _BE // CH
    g = _unpack_pairs(gp[...], nch)
    out[...] = _pack_pairs(_edge_mlp(g, ef[...], w1e, w2, b2, w3, b3), nch)


def _edge_body(gp, efp, w1e, w2, b2, w3, b3, out):
    nch = _BE // CH
    g = _unpack_pairs(gp[...], nch)
    x = _unpack_pairs(efp[...], nch)
    out[...] = _pack_pairs(_edge_mlp(g, x, w1e, w2, b2, w3, b3), nch)


def _node_body(nf, p0, p1, w1a, w1b, b1, w2, b2, w3, b3,
               nw1s, nw1d, nb1, out, ps, pd):
    x = nf[...]
    agg = (p0[0] + p0[1]) + (p1[0] + p1[1])
    h = (jnp.dot(x, w1a[...], preferred_element_type=jnp.float32)
         + jnp.dot(agg, w1b[...], preferred_element_type=jnp.float32) + b1[...])
    h = jnp.maximum(h, 0.0)
    h = jnp.dot(h, w2[...], preferred_element_type=jnp.float32) + b2[...]
    h = jnp.maximum(h, 0.0)
    y = jnp.dot(h, w3[...], preferred_element_type=jnp.float32) + b3[...] + x
    out[...] = y
    ps[...] = jnp.dot(y, nw1s[...], preferred_element_type=jnp.float32) + nb1[...]
    pd[...] = jnp.dot(y, nw1d[...], preferred_element_type=jnp.float32)


def _row_spec(block):
    return pl.BlockSpec((block, D), lambda b: (b, 0))


def _half_spec(block):
    return pl.BlockSpec((block, D // 2), lambda b: (b, 0))


def _w_spec():
    return pl.BlockSpec((D, D), lambda b: (0, 0))


def _b_spec():
    return pl.BlockSpec((1, D), lambda b: (0, 0))


_project_tc = pl.pallas_call(
    _project_body,
    grid=(N // _BN,),
    in_specs=[_row_spec(_BN), _w_spec(), _w_spec(), _b_spec()],
    out_specs=[_row_spec(_BN), _row_spec(_BN)],
    out_shape=[jax.ShapeDtypeStruct((N, D), jnp.float32),
               jax.ShapeDtypeStruct((N, D), jnp.float32)],
)

def _make_edge_tc_first(off_blocks):
    return pl.pallas_call(
        _edge_body_first,
        grid=(ES // _BE,),
        in_specs=[_row_spec(_BE // 2),
                  pl.BlockSpec((_BE, D), lambda b: (b + off_blocks, 0)),
                  _w_spec(), _w_spec(), _b_spec(), _w_spec(), _b_spec()],
        out_specs=_row_spec(_BE // 2),
        out_shape=jax.ShapeDtypeStruct((ES // 2, D), jnp.float32),
    )


_edge_tc = pl.pallas_call(
    _edge_body,
    grid=(ES // _BE,),
    in_specs=[_row_spec(_BE // 2), _row_spec(_BE // 2),
              _w_spec(), _w_spec(), _b_spec(), _w_spec(), _b_spec()],
    out_specs=_row_spec(_BE // 2),
    out_shape=jax.ShapeDtypeStruct((ES // 2, D), jnp.float32),
)
_edge_tc_first = [_make_edge_tc_first(s * (ES // _BE)) for s in range(K_SL)]

_node_tc = pl.pallas_call(
    _node_body,
    grid=(N // _BN,),
    in_specs=[_row_spec(_BN),
              pl.BlockSpec((NC, _BN, D), lambda b: (0, b, 0)),
              pl.BlockSpec((NC, _BN, D), lambda b: (0, b, 0)),
              _w_spec(), _w_spec(), _b_spec(), _w_spec(), _b_spec(),
              _w_spec(), _b_spec(),
              _w_spec(), _w_spec(), _b_spec()],
    out_specs=[_row_spec(_BN), _row_spec(_BN), _row_spec(_BN)],
    out_shape=[jax.ShapeDtypeStruct((N, D), jnp.float32),
               jax.ShapeDtypeStruct((N, D), jnp.float32),
               jax.ShapeDtypeStruct((N, D), jnp.float32)],
)


def _worker_major(idx, s):
    sl = idx[s * ES:(s + 1) * ES]
    pad = NW * RPW * CH - ES
    sl = jnp.concatenate([sl, jnp.zeros((pad,), sl.dtype)])
    return sl.reshape(RPW, NW, CH).transpose(1, 0, 2)


@jax.jit
def kernel(node_features, edge_features, edge_index,
           edge_W1, edge_b1, edge_W2, edge_b2, edge_W3, edge_b3,
           node_W1, node_b1, node_W2, node_b2, node_W3, node_b3):
    gather_sc, scatter_sc = _sc_kernels()
    src3 = [_worker_major(edge_index[0], s) for s in range(K_SL)]
    dst3 = [_worker_major(edge_index[1], s) for s in range(K_SL)]
    zeros = jnp.zeros((CH, D), jnp.float32)

    nf = node_features
    efs = None
    nrounds = edge_W1.shape[0]
    w1 = edge_W1[0]
    ps, pd = _project_tc(nf, w1[:D], w1[D:2 * D], edge_b1[0].reshape(1, D))
    for i in range(nrounds):
        w1 = edge_W1[i]
        ew = (w1[2 * D:], edge_W2[i], edge_b2[i].reshape(1, D),
              edge_W3[i], edge_b3[i].reshape(1, D))
        gs = [gather_sc(ps, pd, src3[s], dst3[s]) for s in range(K_SL)]
        if efs is None:
            efs = [_edge_tc_first[s](gs[s], edge_features, *ew)
                   for s in range(K_SL)]
        else:
            efs = [_edge_tc(gs[s], efs[s], *ew) for s in range(K_SL)]
        parts = [scatter_sc(efs[s], dst3[s], zeros) for s in range(K_SL)]
        nw1 = node_W1[i]
        j = (i + 1) % nrounds
        w1n = edge_W1[j]
        nf, ps, pd = _node_tc(nf, parts[0], parts[1],
                              nw1[:D], nw1[D:], node_b1[i].reshape(1, D),
                              node_W2[i], node_b2[i].reshape(1, D),
                              node_W3[i], node_b3[i].reshape(1, D),
                              w1n[:D], w1n[D:2 * D],
                              edge_b1[j].reshape(1, D))
    return nf
